# Initial kernel scaffold; baseline (speedup 1.0000x reference)
import jax
import jax.numpy as jnp
from jax.experimental import pallas as pl


def kernel(coords, features, W1, b1, Wl1, bl1, gl1w, gl1b, Wp1, bp1, gp1w, gp1b, Wl2, bl2, gl2w, gl2b, Wp2, bp2, gp2w, gp2b, W2, b2, Ws, bs, gsw, gsb):
    raise NotImplementedError("write your pallas kernel here")


# speedup vs baseline: 1.7352x; 1.7352x over previous
import functools

import jax
import jax.numpy as jnp
from jax import lax
from jax.experimental import pallas as pl
from jax.experimental.pallas import tpu as pltpu

N = 16384
KNB = 16
GROUPS = 16
QB = 128


def _knn_body(q_ref, ct_ref, idx_ref, d2_ref):
    q = q_ref[...]
    ct = ct_ref[...]
    qsq = jnp.sum(q * q, axis=1, keepdims=True)
    ksq = jnp.sum(ct * ct, axis=0, keepdims=True)
    d2 = qsq + ksq - 2.0 * jnp.dot(q, ct, preferred_element_type=jnp.float32)
    d2 = jnp.maximum(d2, 0.0)
    iota = lax.broadcasted_iota(jnp.int32, (QB, N), 1)
    for s in range(KNB):
        m = jnp.min(d2, axis=1, keepdims=True)
        am = jnp.min(jnp.where(d2 == m, iota, N), axis=1, keepdims=True)
        idx_ref[:, s] = am[:, 0]
        d2_ref[:, s] = m[:, 0]
        d2 = jnp.where(iota == am, 1e30, d2)


def _knn(coords):
    ct = coords.T
    grid = N // QB
    idx, d2 = pl.pallas_call(
        _knn_body,
        grid=(grid,),
        in_specs=[
            pl.BlockSpec((QB, 3), lambda i: (i, 0)),
            pl.BlockSpec((3, N), lambda i: (0, 0)),
        ],
        out_specs=[
            pl.BlockSpec((QB, KNB), lambda i: (i, 0)),
            pl.BlockSpec((QB, KNB), lambda i: (i, 0)),
        ],
        out_shape=[
            jax.ShapeDtypeStruct((N, KNB), jnp.int32),
            jax.ShapeDtypeStruct((N, KNB), jnp.float32),
        ],
    )(coords, ct)
    return idx, d2


def _smlp(x, W, b):
    return jnp.einsum('oc,bcnk->bonk', W, x) + b[None, :, None, None]


def _gnorm(x, w, b, groups=GROUPS, eps=1e-6):
    Bb, C, Nn, Kk = x.shape
    xg = x.reshape(Bb, groups, C // groups, Nn, Kk)
    mean = jnp.mean(xg, axis=(2, 3, 4), keepdims=True)
    var = jnp.var(xg, axis=(2, 3, 4), keepdims=True)
    xg = (xg - mean) / jnp.sqrt(var + eps)
    x = xg.reshape(Bb, C, Nn, Kk)
    return x * w[None, :, None, None] + b[None, :, None, None]


def _lse(coords, features, idx, dist, W, b, gw, gb):
    Bb, Nn, Kk = idx.shape
    ec = jnp.transpose(coords, (0, 2, 1))[:, :, :, None]
    ec = jnp.broadcast_to(ec, (Bb, 3, Nn, Kk))
    nc = jax.vmap(lambda cc, ii: cc[ii])(coords, idx)
    nc = jnp.transpose(nc, (0, 3, 1, 2))
    f = features[..., 0]
    nf = jax.vmap(lambda fb, ib: fb[:, ib])(f, idx)
    concat = jnp.concatenate([ec, nc, ec - nc, dist[:, None, :, :]], axis=1)
    h = jax.nn.relu(_gnorm(_smlp(concat, W, b), gw, gb))
    return jnp.concatenate([h, nf], axis=1)


def _att_pool(x, W, b, gw, gb):
    m = jnp.mean(x, axis=-1, keepdims=True)
    return jax.nn.relu(_gnorm(_smlp(m, W, b), gw, gb))


def kernel(coords, features, W1, b1, Wl1, bl1, gl1w, gl1b, Wp1, bp1, gp1w, gp1b, Wl2, bl2, gl2w, gl2b, Wp2, bp2, gp2w, gp2b, W2, b2, Ws, bs, gsw, gsb):
    idx0, d20 = _knn(coords[0])
    idx = idx0[None]
    dist = jnp.sqrt(d20)[None]
    x = jax.nn.leaky_relu(_smlp(features, W1, b1), 0.2)
    x = _lse(coords, x, idx, dist, Wl1, bl1, gl1w, gl1b)
    x = _att_pool(x, Wp1, bp1, gp1w, gp1b)
    x = _lse(coords, x, idx, dist, Wl2, bl2, gl2w, gl2b)
    x = _att_pool(x, Wp2, bp2, gp2w, gp2b)
    shortcut = _gnorm(_smlp(features, Ws, bs), gsw, gsb)
    return jax.nn.leaky_relu(_smlp(x, W2, b2) + shortcut, 0.01)


# speedup vs baseline: 4.1718x; 2.4042x over previous
import functools

import jax
import jax.numpy as jnp
from jax import lax
from jax.experimental import pallas as pl
from jax.experimental.pallas import tpu as pltpu

N = 16384
KNB = 16
NK = N * KNB
EPS = 1e-6
QB = 128
AB = 256
CB = 512


def _knn_body(q_ref, ct_ref, idx_ref, dist_ref):
    q = q_ref[...]
    ct = ct_ref[...]
    qsq = jnp.sum(q * q, axis=1, keepdims=True)
    ksq = jnp.sum(ct * ct, axis=0, keepdims=True)
    d2 = qsq + ksq - 2.0 * jnp.dot(q, ct, preferred_element_type=jnp.float32)
    d2 = jnp.maximum(d2, 0.0)
    iota = lax.broadcasted_iota(jnp.int32, (QB, N), 1)
    for s in range(KNB):
        m = jnp.min(d2, axis=1, keepdims=True)
        am = jnp.min(jnp.where(d2 == m, iota, N), axis=1, keepdims=True)
        idx_ref[:, s] = am[:, 0]
        dist_ref[:, s] = jnp.sqrt(m[:, 0])
        d2 = jnp.where(iota == am, 1e30, d2)


def _knn(coords):
    ct = coords.T
    idx, dist = pl.pallas_call(
        _knn_body,
        grid=(N // QB,),
        in_specs=[
            pl.BlockSpec((QB, 3), lambda i: (i, 0)),
            pl.BlockSpec((3, N), lambda i: (0, 0)),
        ],
        out_specs=[
            pl.BlockSpec((QB, KNB), lambda i: (i, 0)),
            pl.BlockSpec((QB, KNB), lambda i: (i, 0)),
        ],
        out_shape=[
            jax.ShapeDtypeStruct((N, KNB), jnp.int32),
            jax.ShapeDtypeStruct((N, KNB), jnp.float32),
        ],
    )(coords, ct)
    return idx, dist


def _p1_body(coords_ref, featT_ref, vw_ref, uw_ref, f1w_ref, b1_ref,
             wsT_ref, t_ref, u_ref, ssst_ref):
    i = pl.program_id(0)
    c = coords_ref[...]
    ft = featT_ref[...]
    v = jnp.dot(c, vw_ref[...], preferred_element_type=jnp.float32)
    f1p = jnp.dot(ft, f1w_ref[...], preferred_element_type=jnp.float32)
    f1p = f1p + b1_ref[0:1, :]
    f1 = jnp.where(f1p > 0, f1p, 0.2 * f1p)
    t_ref[...] = jnp.concatenate([v, f1], axis=1)
    u = jnp.dot(c, uw_ref[...], preferred_element_type=jnp.float32)
    u_ref[...] = jnp.concatenate([u, jnp.zeros_like(f1)], axis=1)
    ss = jnp.dot(ft, wsT_ref[...], preferred_element_type=jnp.float32)

    @pl.when(i == 0)
    def _():
        ssst_ref[...] = jnp.zeros_like(ssst_ref)

    s1 = jnp.sum(ss, axis=0)
    s2 = jnp.sum(ss * ss, axis=0)
    pad = jnp.zeros((6, 64), jnp.float32)
    ssst_ref[...] += jnp.concatenate([s1[None], s2[None], pad], axis=0)


def _p1(coords, featT, vw, uw, f1w, b1row, wsT):
    PB = 2048
    return pl.pallas_call(
        _p1_body,
        grid=(N // PB,),
        in_specs=[
            pl.BlockSpec((PB, 3), lambda i: (i, 0)),
            pl.BlockSpec((PB, 8), lambda i: (i, 0)),
            pl.BlockSpec((3, 32), lambda i: (0, 0)),
            pl.BlockSpec((3, 32), lambda i: (0, 0)),
            pl.BlockSpec((8, 16), lambda i: (0, 0)),
            pl.BlockSpec((8, 16), lambda i: (0, 0)),
            pl.BlockSpec((8, 64), lambda i: (0, 0)),
        ],
        out_specs=[
            pl.BlockSpec((PB, 48), lambda i: (i, 0)),
            pl.BlockSpec((PB, 48), lambda i: (i, 0)),
            pl.BlockSpec((8, 64), lambda i: (0, 0)),
        ],
        out_shape=[
            jax.ShapeDtypeStruct((N, 48), jnp.float32),
            jax.ShapeDtypeStruct((N, 48), jnp.float32),
            jax.ShapeDtypeStruct((8, 64), jnp.float32),
        ],
    )(coords, featT, vw, uw, f1w, b1row, wsT)


def _passA_body(g_ref, u_ref, d_ref, wall_ref, tst_ref, nf1_ref):
    i = pl.program_id(0)
    t3 = (g_ref[...] + u_ref[...][:, None, :]
          + d_ref[...][:, :, None] * wall_ref[0:1, :][None, :, :])

    @pl.when(i == 0)
    def _():
        tst_ref[...] = jnp.zeros_like(tst_ref)

    s1 = jnp.sum(t3, axis=(0, 1))
    s2 = jnp.sum(t3 * t3, axis=(0, 1))
    pad = jnp.zeros((6, 48), jnp.float32)
    tst_ref[...] += jnp.concatenate([s1[None], s2[None], pad], axis=0)
    nf1_ref[...] = jnp.mean(t3[:, :, 32:48], axis=1)


def _passA(g3, u48, dist, wall):
    return pl.pallas_call(
        _passA_body,
        grid=(N // AB,),
        in_specs=[
            pl.BlockSpec((AB, KNB, 48), lambda i: (i, 0, 0)),
            pl.BlockSpec((AB, 48), lambda i: (i, 0)),
            pl.BlockSpec((AB, KNB), lambda i: (i, 0)),
            pl.BlockSpec((8, 48), lambda i: (0, 0)),
        ],
        out_specs=[
            pl.BlockSpec((8, 48), lambda i: (0, 0)),
            pl.BlockSpec((AB, 16), lambda i: (i, 0)),
        ],
        out_shape=[
            jax.ShapeDtypeStruct((8, 48), jnp.float32),
            jax.ShapeDtypeStruct((N, 16), jnp.float32),
        ],
    )(g3, u48, dist, wall)


def _passB1_body(g_ref, u_ref, d_ref, wall_ref, aff_ref, nf1_ref, wp1_ref,
                 y1_ref, mh2_ref, yst_ref):
    i = pl.program_id(0)
    t3 = (g_ref[...] + u_ref[...][:, None, :]
          + d_ref[...][:, :, None] * wall_ref[0:1, :][None, :, :])
    h3 = t3 * aff_ref[0:1, :][None, :, :] + aff_ref[1:2, :][None, :, :]
    h3 = jnp.maximum(h3, 0.0)
    mh = jnp.mean(h3, axis=1)
    pooled1 = jnp.concatenate([mh[:, 0:16], nf1_ref[...]], axis=1)
    y1 = jnp.dot(pooled1, wp1_ref[...], preferred_element_type=jnp.float32)
    y1_ref[...] = y1
    mh2_ref[...] = mh[:, 16:32]

    @pl.when(i == 0)
    def _():
        yst_ref[...] = jnp.zeros_like(yst_ref)

    s1 = jnp.sum(y1, axis=0)
    s2 = jnp.sum(y1 * y1, axis=0)
    pad = jnp.zeros((6, 16), jnp.float32)
    yst_ref[...] += jnp.concatenate([s1[None], s2[None], pad], axis=0)


def _passB1(g3, u48, dist, wall, aff48, nf1, wp1T):
    return pl.pallas_call(
        _passB1_body,
        grid=(N // AB,),
        in_specs=[
            pl.BlockSpec((AB, KNB, 48), lambda i: (i, 0, 0)),
            pl.BlockSpec((AB, 48), lambda i: (i, 0)),
            pl.BlockSpec((AB, KNB), lambda i: (i, 0)),
            pl.BlockSpec((8, 48), lambda i: (0, 0)),
            pl.BlockSpec((8, 48), lambda i: (0, 0)),
            pl.BlockSpec((AB, 16), lambda i: (i, 0)),
            pl.BlockSpec((32, 16), lambda i: (0, 0)),
        ],
        out_specs=[
            pl.BlockSpec((AB, 16), lambda i: (i, 0)),
            pl.BlockSpec((AB, 16), lambda i: (i, 0)),
            pl.BlockSpec((8, 16), lambda i: (0, 0)),
        ],
        out_shape=[
            jax.ShapeDtypeStruct((N, 16), jnp.float32),
            jax.ShapeDtypeStruct((N, 16), jnp.float32),
            jax.ShapeDtypeStruct((8, 16), jnp.float32),
        ],
    )(g3, u48, dist, wall, aff48, nf1, wp1T)


def _passB3_body(mh2_ref, nf2_ref, wp2_ref, y2_ref, yst_ref):
    i = pl.program_id(0)
    pooled2 = jnp.concatenate([mh2_ref[...], nf2_ref[...]], axis=1)
    y2 = jnp.dot(pooled2, wp2_ref[...], preferred_element_type=jnp.float32)
    y2_ref[...] = y2

    @pl.when(i == 0)
    def _():
        yst_ref[...] = jnp.zeros_like(yst_ref)

    s1 = jnp.sum(y2, axis=0)
    s2 = jnp.sum(y2 * y2, axis=0)
    pad = jnp.zeros((6, 32), jnp.float32)
    yst_ref[...] += jnp.concatenate([s1[None], s2[None], pad], axis=0)


def _passB3(mh2, nf2, wp2T):
    return pl.pallas_call(
        _passB3_body,
        grid=(N // CB,),
        in_specs=[
            pl.BlockSpec((CB, 16), lambda i: (i, 0)),
            pl.BlockSpec((CB, 16), lambda i: (i, 0)),
            pl.BlockSpec((32, 32), lambda i: (0, 0)),
        ],
        out_specs=[
            pl.BlockSpec((CB, 32), lambda i: (i, 0)),
            pl.BlockSpec((8, 32), lambda i: (0, 0)),
        ],
        out_shape=[
            jax.ShapeDtypeStruct((N, 32), jnp.float32),
            jax.ShapeDtypeStruct((8, 32), jnp.float32),
        ],
    )(mh2, nf2, wp2T)


def _passB4_body(y2_ref, f8_ref, aff2_ref, ws_ref, w2_ref, saff_ref, b2_ref,
                 out_ref):
    y2 = y2_ref[...]
    x2 = jnp.maximum(y2 * aff2_ref[0:1, :] + aff2_ref[1:2, :], 0.0)
    ss = jnp.dot(ws_ref[...], f8_ref[...], preferred_element_type=jnp.float32)
    ssn = ss * saff_ref[:, 0:1] + saff_ref[:, 1:2]
    o = lax.dot_general(w2_ref[...], x2, (((1,), (1,)), ((), ())),
                        preferred_element_type=jnp.float32)
    o = o + b2_ref[...] + ssn
    out_ref[...] = jnp.where(o > 0, o, 0.01 * o)


def _passB4(y2, feat8, aff2, ws, w2, saff, b2col):
    return pl.pallas_call(
        _passB4_body,
        grid=(N // CB,),
        in_specs=[
            pl.BlockSpec((CB, 32), lambda i: (i, 0)),
            pl.BlockSpec((8, CB), lambda i: (0, i)),
            pl.BlockSpec((8, 32), lambda i: (0, 0)),
            pl.BlockSpec((64, 8), lambda i: (0, 0)),
            pl.BlockSpec((64, 32), lambda i: (0, 0)),
            pl.BlockSpec((64, 8), lambda i: (0, 0)),
            pl.BlockSpec((64, 1), lambda i: (0, 0)),
        ],
        out_specs=pl.BlockSpec((64, CB), lambda i: (0, i)),
        out_shape=jax.ShapeDtypeStruct((64, N), jnp.float32),
    )(y2, feat8, aff2, ws, w2, saff, b2col)


def _pad8(row):
    return jnp.concatenate([row[None, :], jnp.zeros((7, row.shape[0]), row.dtype)], 0)


def _stats_to_aff(s1, s2, count, gw, gb, group_size):
    C = s1.shape[0]
    g1 = s1.reshape(C // group_size, group_size).sum(1)
    g2 = s2.reshape(C // group_size, group_size).sum(1)
    m = g1 / (count * group_size)
    var = g2 / (count * group_size) - m * m
    inv = 1.0 / jnp.sqrt(var + EPS)
    m = jnp.repeat(m, group_size)
    inv = jnp.repeat(inv, group_size)
    scale = gw * inv
    shift = gb - m * scale
    return scale, shift


def kernel(coords, features, W1, b1, Wl1, bl1, gl1w, gl1b, Wp1, bp1, gp1w, gp1b, Wl2, bl2, gl2w, gl2b, Wp2, bp2, gp2w, gp2b, W2, b2, Ws, bs, gsw, gsb):
    c0 = coords[0]
    feat8 = features[0, :, :, 0]
    featT = feat8.T

    idx, dist = _knn(c0)
    idxf = idx.reshape(-1)

    A1 = Wl1[:, 0:3] + Wl1[:, 6:9]
    B1m = Wl1[:, 3:6] - Wl1[:, 6:9]
    w1 = Wl1[:, 9]
    A2 = Wl2[:, 0:3] + Wl2[:, 6:9]
    B2m = Wl2[:, 3:6] - Wl2[:, 6:9]
    w2 = Wl2[:, 9]
    vw = jnp.concatenate([B1m.T, B2m.T], axis=1)
    uw = jnp.concatenate([A1.T, A2.T], axis=1)
    wall = _pad8(jnp.concatenate([w1, w2, jnp.zeros(16, jnp.float32)]))

    T, U48, ssst = _p1(c0, featT, vw, uw, W1.T, _pad8(b1), Ws.T)

    g3 = jnp.take(T, idxf, axis=0).reshape(N, KNB, 48)

    tst, nf1 = _passA(g3, U48, dist, wall)

    gl12w = jnp.concatenate([gl1w, gl2w, jnp.zeros(16, jnp.float32)])
    gl12b = jnp.concatenate([gl1b, gl2b, jnp.zeros(16, jnp.float32)])
    sc48, sh48 = _stats_to_aff(tst[0], tst[1], float(NK), gl12w, gl12b, 1)
    aff48 = jnp.concatenate([sc48[None], sh48[None],
                             jnp.zeros((6, 48), jnp.float32)], 0)

    y1, mh2, y1st = _passB1(g3, U48, dist, wall, aff48, nf1, Wp1.T)

    sc1, shf1 = _stats_to_aff(y1st[0], y1st[1], float(N), gp1w, gp1b, 1)
    x1 = jnp.maximum(y1 * sc1[None, :] + shf1[None, :], 0.0)
    nf2 = jnp.take(x1, idxf, axis=0).reshape(N, KNB, 16).mean(axis=1)

    y2, y2st = _passB3(mh2, nf2, Wp2.T)

    sc2, shf2 = _stats_to_aff(y2st[0], y2st[1], float(N), gp2w, gp2b, 2)
    aff2 = jnp.concatenate([sc2[None], shf2[None],
                            jnp.zeros((6, 32), jnp.float32)], 0)
    scs, shs = _stats_to_aff(ssst[0], ssst[1], float(N), gsw, gsb, 4)
    saff = jnp.concatenate([scs[:, None], shs[:, None],
                            jnp.zeros((64, 6), jnp.float32)], 1)

    out = _passB4(y2, feat8, aff2, Ws, W2, saff, b2[:, None])
    return out[None, :, :, None]


# speedup vs baseline: 5.1753x; 1.2405x over previous
import functools

import jax
import jax.numpy as jnp
from jax import lax
from jax.experimental import pallas as pl
from jax.experimental.pallas import tpu as pltpu
from jax.experimental.pallas import tpu_sc as plsc

N = 16384
KNB = 16
NK = N * KNB
EPS = 1e-6
QB = 128
AB = 256
CB = 512


def _knn_body(q_ref, ct_ref, idx_ref, dist_ref):
    q = q_ref[...]
    ct = ct_ref[...]
    qsq = jnp.sum(q * q, axis=1, keepdims=True)
    ksq = jnp.sum(ct * ct, axis=0, keepdims=True)
    d2 = qsq + ksq - 2.0 * jnp.dot(q, ct, preferred_element_type=jnp.float32)
    d2 = jnp.maximum(d2, 0.0)
    iota = lax.broadcasted_iota(jnp.int32, (QB, N), 1)
    for s in range(KNB):
        m = jnp.min(d2, axis=1, keepdims=True)
        am = jnp.min(jnp.where(d2 == m, iota, N), axis=1, keepdims=True)
        idx_ref[:, s] = am[:, 0]
        dist_ref[:, s] = jnp.sqrt(m[:, 0])
        d2 = jnp.where(iota == am, 1e30, d2)


def _knn(coords):
    ct = coords.T
    idx, dist = pl.pallas_call(
        _knn_body,
        grid=(N // QB,),
        in_specs=[
            pl.BlockSpec((QB, 3), lambda i: (i, 0)),
            pl.BlockSpec((3, N), lambda i: (0, 0)),
        ],
        out_specs=[
            pl.BlockSpec((QB, KNB), lambda i: (i, 0)),
            pl.BlockSpec((QB, KNB), lambda i: (i, 0)),
        ],
        out_shape=[
            jax.ShapeDtypeStruct((N, KNB), jnp.int32),
            jax.ShapeDtypeStruct((N, KNB), jnp.float32),
        ],
    )(coords, ct)
    return idx, dist


_NW = 32
_CHUNK = 128
_PER_W = NK // _NW


def _sc_gather_t(table, idxf):
    mesh = plsc.VectorSubcoreMesh(core_axis_name="c", subcore_axis_name="s")

    @functools.partial(
        pl.kernel, mesh=mesh,
        out_type=jax.ShapeDtypeStruct((NK, 128), jnp.float32),
        scratch_types=[
            pltpu.VMEM((_CHUNK,), jnp.int32),
            pltpu.VMEM((_CHUNK, 128), jnp.float32),
            pltpu.SemaphoreType.DMA,
        ],
    )
    def k(table_hbm, idx_hbm, out_hbm, idx_c, rows, sem):
        wid = lax.axis_index("s") * 2 + lax.axis_index("c")
        base = wid * _PER_W
        nch = _PER_W // _CHUNK

        def body(c, _):
            off = pl.multiple_of(base + c * _CHUNK, _CHUNK)
            pltpu.sync_copy(idx_hbm.at[pl.ds(off, _CHUNK)], idx_c)
            pltpu.async_copy(table_hbm.at[idx_c], rows, sem).wait()
            pltpu.sync_copy(rows, out_hbm.at[pl.ds(off, _CHUNK)])
            return _

        lax.fori_loop(0, nch, body, 0)

    return k(table, idxf)


def _sc_gather_nf2(y1p, idxf, sc16, sh16):
    mesh = plsc.VectorSubcoreMesh(core_axis_name="c", subcore_axis_name="s")
    aff = jnp.concatenate([sc16[None], sh16[None]], axis=0)

    @functools.partial(
        pl.kernel, mesh=mesh,
        out_type=jax.ShapeDtypeStruct((N, 16), jnp.float32),
        scratch_types=[
            pltpu.VMEM((_CHUNK,), jnp.int32),
            pltpu.VMEM((_CHUNK, 128), jnp.float32),
            pltpu.VMEM((2, 16), jnp.float32),
            pltpu.VMEM((8, 16), jnp.float32),
            pltpu.SemaphoreType.DMA,
        ],
    )
    def k(y1_hbm, idx_hbm, aff_hbm, out_hbm, idx_c, rows, aff_v, acc_v, sem):
        wid = lax.axis_index("s") * 2 + lax.axis_index("c")
        base = wid * _PER_W
        pltpu.sync_copy(aff_hbm, aff_v)
        nch = _PER_W // _CHUNK

        def body(c, _):
            off = pl.multiple_of(base + c * _CHUNK, _CHUNK)
            pltpu.sync_copy(idx_hbm.at[pl.ds(off, _CHUNK)], idx_c)
            pltpu.async_copy(y1_hbm.at[idx_c], rows, sem).wait()
            sc = aff_v[0, :]
            sh = aff_v[1, :]
            for i in range(_CHUNK // KNB):
                acc = jnp.zeros((16,), jnp.float32)
                for j in range(KNB):
                    r = rows[i * KNB + j, 0:16]
                    acc = acc + jnp.maximum(r * sc + sh, 0.0)
                acc_v[i, :] = acc * (1.0 / KNB)
            out_rowbase = pl.multiple_of((base + c * _CHUNK) // KNB, 8)
            pltpu.sync_copy(acc_v, out_hbm.at[pl.ds(out_rowbase, 8)])
            return _

        lax.fori_loop(0, nch, body, 0)

    return k(y1p, idxf, aff)


def _p1_body(coords_ref, featT_ref, vw_ref, uw_ref, f1w_ref, b1_ref,
             wsT_ref, t_ref, u_ref, ssst_ref):
    i = pl.program_id(0)
    c = coords_ref[...]
    ft = featT_ref[...]
    v = jnp.dot(c, vw_ref[...], preferred_element_type=jnp.float32)
    f1p = jnp.dot(ft, f1w_ref[...], preferred_element_type=jnp.float32)
    f1p = f1p + b1_ref[0:1, :]
    f1 = jnp.where(f1p > 0, f1p, 0.2 * f1p)
    t_ref[...] = jnp.concatenate(
        [v, f1, jnp.zeros((v.shape[0], 80), jnp.float32)], axis=1)
    u = jnp.dot(c, uw_ref[...], preferred_element_type=jnp.float32)
    u_ref[...] = jnp.concatenate([u, jnp.zeros_like(f1)], axis=1)
    ss = jnp.dot(ft, wsT_ref[...], preferred_element_type=jnp.float32)

    @pl.when(i == 0)
    def _():
        ssst_ref[...] = jnp.zeros_like(ssst_ref)

    s1 = jnp.sum(ss, axis=0)
    s2 = jnp.sum(ss * ss, axis=0)
    pad = jnp.zeros((6, 64), jnp.float32)
    ssst_ref[...] += jnp.concatenate([s1[None], s2[None], pad], axis=0)


def _p1(coords, featT, vw, uw, f1w, b1row, wsT):
    PB = 2048
    return pl.pallas_call(
        _p1_body,
        grid=(N // PB,),
        in_specs=[
            pl.BlockSpec((PB, 3), lambda i: (i, 0)),
            pl.BlockSpec((PB, 8), lambda i: (i, 0)),
            pl.BlockSpec((3, 32), lambda i: (0, 0)),
            pl.BlockSpec((3, 32), lambda i: (0, 0)),
            pl.BlockSpec((8, 16), lambda i: (0, 0)),
            pl.BlockSpec((8, 16), lambda i: (0, 0)),
            pl.BlockSpec((8, 64), lambda i: (0, 0)),
        ],
        out_specs=[
            pl.BlockSpec((PB, 128), lambda i: (i, 0)),
            pl.BlockSpec((PB, 48), lambda i: (i, 0)),
            pl.BlockSpec((8, 64), lambda i: (0, 0)),
        ],
        out_shape=[
            jax.ShapeDtypeStruct((N, 128), jnp.float32),
            jax.ShapeDtypeStruct((N, 48), jnp.float32),
            jax.ShapeDtypeStruct((8, 64), jnp.float32),
        ],
    )(coords, featT, vw, uw, f1w, b1row, wsT)


def _passA_body(g_ref, u_ref, d_ref, wall_ref, tst_ref, nf1_ref):
    i = pl.program_id(0)
    t3 = (g_ref[...][:, :, 0:48] + u_ref[...][:, None, :]
          + d_ref[...][:, :, None] * wall_ref[0:1, :][None, :, :])

    @pl.when(i == 0)
    def _():
        tst_ref[...] = jnp.zeros_like(tst_ref)

    s1 = jnp.sum(t3, axis=(0, 1))
    s2 = jnp.sum(t3 * t3, axis=(0, 1))
    pad = jnp.zeros((6, 48), jnp.float32)
    tst_ref[...] += jnp.concatenate([s1[None], s2[None], pad], axis=0)
    nf1_ref[...] = jnp.mean(t3[:, :, 32:48], axis=1)


def _passA(g3, u48, dist, wall):
    return pl.pallas_call(
        _passA_body,
        grid=(N // AB,),
        in_specs=[
            pl.BlockSpec((AB, KNB, 128), lambda i: (i, 0, 0)),
            pl.BlockSpec((AB, 48), lambda i: (i, 0)),
            pl.BlockSpec((AB, KNB), lambda i: (i, 0)),
            pl.BlockSpec((8, 48), lambda i: (0, 0)),
        ],
        out_specs=[
            pl.BlockSpec((8, 48), lambda i: (0, 0)),
            pl.BlockSpec((AB, 16), lambda i: (i, 0)),
        ],
        out_shape=[
            jax.ShapeDtypeStruct((8, 48), jnp.float32),
            jax.ShapeDtypeStruct((N, 16), jnp.float32),
        ],
    )(g3, u48, dist, wall)


def _passB1_body(g_ref, u_ref, d_ref, wall_ref, aff_ref, nf1_ref, wp1_ref,
                 y1_ref, mh2_ref, yst_ref):
    i = pl.program_id(0)
    t3 = (g_ref[...][:, :, 0:48] + u_ref[...][:, None, :]
          + d_ref[...][:, :, None] * wall_ref[0:1, :][None, :, :])
    h3 = t3 * aff_ref[0:1, :][None, :, :] + aff_ref[1:2, :][None, :, :]
    h3 = jnp.maximum(h3, 0.0)
    mh = jnp.mean(h3, axis=1)
    pooled1 = jnp.concatenate([mh[:, 0:16], nf1_ref[...]], axis=1)
    y1 = jnp.dot(pooled1, wp1_ref[...], preferred_element_type=jnp.float32)
    y1_ref[...] = y1
    mh2_ref[...] = mh[:, 16:32]

    @pl.when(i == 0)
    def _():
        yst_ref[...] = jnp.zeros_like(yst_ref)

    s1 = jnp.sum(y1, axis=0)
    s2 = jnp.sum(y1 * y1, axis=0)
    pad = jnp.zeros((6, 16), jnp.float32)
    yst_ref[...] += jnp.concatenate([s1[None], s2[None], pad], axis=0)


def _passB1(g3, u48, dist, wall, aff48, nf1, wp1T):
    return pl.pallas_call(
        _passB1_body,
        grid=(N // AB,),
        in_specs=[
            pl.BlockSpec((AB, KNB, 128), lambda i: (i, 0, 0)),
            pl.BlockSpec((AB, 48), lambda i: (i, 0)),
            pl.BlockSpec((AB, KNB), lambda i: (i, 0)),
            pl.BlockSpec((8, 48), lambda i: (0, 0)),
            pl.BlockSpec((8, 48), lambda i: (0, 0)),
            pl.BlockSpec((AB, 16), lambda i: (i, 0)),
            pl.BlockSpec((32, 16), lambda i: (0, 0)),
        ],
        out_specs=[
            pl.BlockSpec((AB, 16), lambda i: (i, 0)),
            pl.BlockSpec((AB, 16), lambda i: (i, 0)),
            pl.BlockSpec((8, 16), lambda i: (0, 0)),
        ],
        out_shape=[
            jax.ShapeDtypeStruct((N, 16), jnp.float32),
            jax.ShapeDtypeStruct((N, 16), jnp.float32),
            jax.ShapeDtypeStruct((8, 16), jnp.float32),
        ],
    )(g3, u48, dist, wall, aff48, nf1, wp1T)


def _passB3_body(mh2_ref, nf2_ref, wp2_ref, y2_ref, yst_ref):
    i = pl.program_id(0)
    pooled2 = jnp.concatenate([mh2_ref[...], nf2_ref[...]], axis=1)
    y2 = jnp.dot(pooled2, wp2_ref[...], preferred_element_type=jnp.float32)
    y2_ref[...] = y2

    @pl.when(i == 0)
    def _():
        yst_ref[...] = jnp.zeros_like(yst_ref)

    s1 = jnp.sum(y2, axis=0)
    s2 = jnp.sum(y2 * y2, axis=0)
    pad = jnp.zeros((6, 32), jnp.float32)
    yst_ref[...] += jnp.concatenate([s1[None], s2[None], pad], axis=0)


def _passB3(mh2, nf2, wp2T):
    return pl.pallas_call(
        _passB3_body,
        grid=(N // CB,),
        in_specs=[
            pl.BlockSpec((CB, 16), lambda i: (i, 0)),
            pl.BlockSpec((CB, 16), lambda i: (i, 0)),
            pl.BlockSpec((32, 32), lambda i: (0, 0)),
        ],
        out_specs=[
            pl.BlockSpec((CB, 32), lambda i: (i, 0)),
            pl.BlockSpec((8, 32), lambda i: (0, 0)),
        ],
        out_shape=[
            jax.ShapeDtypeStruct((N, 32), jnp.float32),
            jax.ShapeDtypeStruct((8, 32), jnp.float32),
        ],
    )(mh2, nf2, wp2T)


def _passB4_body(y2_ref, f8_ref, aff2_ref, ws_ref, w2_ref, saff_ref, b2_ref,
                 out_ref):
    y2 = y2_ref[...]
    x2 = jnp.maximum(y2 * aff2_ref[0:1, :] + aff2_ref[1:2, :], 0.0)
    ss = jnp.dot(ws_ref[...], f8_ref[...], preferred_element_type=jnp.float32)
    ssn = ss * saff_ref[:, 0:1] + saff_ref[:, 1:2]
    o = lax.dot_general(w2_ref[...], x2, (((1,), (1,)), ((), ())),
                        preferred_element_type=jnp.float32)
    o = o + b2_ref[...] + ssn
    out_ref[...] = jnp.where(o > 0, o, 0.01 * o)


def _passB4(y2, feat8, aff2, ws, w2, saff, b2col):
    return pl.pallas_call(
        _passB4_body,
        grid=(N // CB,),
        in_specs=[
            pl.BlockSpec((CB, 32), lambda i: (i, 0)),
            pl.BlockSpec((8, CB), lambda i: (0, i)),
            pl.BlockSpec((8, 32), lambda i: (0, 0)),
            pl.BlockSpec((64, 8), lambda i: (0, 0)),
            pl.BlockSpec((64, 32), lambda i: (0, 0)),
            pl.BlockSpec((64, 8), lambda i: (0, 0)),
            pl.BlockSpec((64, 1), lambda i: (0, 0)),
        ],
        out_specs=pl.BlockSpec((64, CB), lambda i: (0, i)),
        out_shape=jax.ShapeDtypeStruct((64, N), jnp.float32),
    )(y2, feat8, aff2, ws, w2, saff, b2col)


def _pad8(row):
    return jnp.concatenate([row[None, :], jnp.zeros((7, row.shape[0]), row.dtype)], 0)


def _stats_to_aff(s1, s2, count, gw, gb, group_size):
    C = s1.shape[0]
    g1 = s1.reshape(C // group_size, group_size).sum(1)
    g2 = s2.reshape(C // group_size, group_size).sum(1)
    m = g1 / (count * group_size)
    var = g2 / (count * group_size) - m * m
    inv = 1.0 / jnp.sqrt(var + EPS)
    m = jnp.repeat(m, group_size)
    inv = jnp.repeat(inv, group_size)
    scale = gw * inv
    shift = gb - m * scale
    return scale, shift


def kernel(coords, features, W1, b1, Wl1, bl1, gl1w, gl1b, Wp1, bp1, gp1w, gp1b, Wl2, bl2, gl2w, gl2b, Wp2, bp2, gp2w, gp2b, W2, b2, Ws, bs, gsw, gsb):
    c0 = coords[0]
    feat8 = features[0, :, :, 0]
    featT = feat8.T

    idx, dist = _knn(c0)
    idxf = idx.reshape(-1)

    A1 = Wl1[:, 0:3] + Wl1[:, 6:9]
    B1m = Wl1[:, 3:6] - Wl1[:, 6:9]
    w1 = Wl1[:, 9]
    A2 = Wl2[:, 0:3] + Wl2[:, 6:9]
    B2m = Wl2[:, 3:6] - Wl2[:, 6:9]
    w2 = Wl2[:, 9]
    vw = jnp.concatenate([B1m.T, B2m.T], axis=1)
    uw = jnp.concatenate([A1.T, A2.T], axis=1)
    wall = _pad8(jnp.concatenate([w1, w2, jnp.zeros(16, jnp.float32)]))

    T, U48, ssst = _p1(c0, featT, vw, uw, W1.T, _pad8(b1), Ws.T)

    g3 = _sc_gather_t(T, idxf).reshape(N, KNB, 128)

    tst, nf1 = _passA(g3, U48, dist, wall)

    gl12w = jnp.concatenate([gl1w, gl2w, jnp.zeros(16, jnp.float32)])
    gl12b = jnp.concatenate([gl1b, gl2b, jnp.zeros(16, jnp.float32)])
    sc48, sh48 = _stats_to_aff(tst[0], tst[1], float(NK), gl12w, gl12b, 1)
    aff48 = jnp.concatenate([sc48[None], sh48[None],
                             jnp.zeros((6, 48), jnp.float32)], 0)

    y1, mh2, y1st = _passB1(g3, U48, dist, wall, aff48, nf1, Wp1.T)

    sc1, shf1 = _stats_to_aff(y1st[0], y1st[1], float(N), gp1w, gp1b, 1)
    y1p = jnp.concatenate([y1, jnp.zeros((N, 112), jnp.float32)], axis=1)
    nf2 = _sc_gather_nf2(y1p, idxf, sc1, shf1)

    y2, y2st = _passB3(mh2, nf2, Wp2.T)

    sc2, shf2 = _stats_to_aff(y2st[0], y2st[1], float(N), gp2w, gp2b, 2)
    aff2 = jnp.concatenate([sc2[None], shf2[None],
                            jnp.zeros((6, 32), jnp.float32)], 0)
    scs, shs = _stats_to_aff(ssst[0], ssst[1], float(N), gsw, gsb, 4)
    saff = jnp.concatenate([scs[:, None], shs[:, None],
                            jnp.zeros((64, 6), jnp.float32)], 1)

    out = _passB4(y2, feat8, aff2, Ws, W2, saff, b2[:, None])
    return out[None, :, :, None]


# speedup vs baseline: 7.0996x; 1.3718x over previous
import functools

import jax
import jax.numpy as jnp
from jax import lax
from jax.experimental import pallas as pl
from jax.experimental.pallas import tpu as pltpu
from jax.experimental.pallas import tpu_sc as plsc

N = 16384
KNB = 16
NK = N * KNB
EPS = 1e-6
QB = 128
AB = 256
CB = 512


_NG = 128
_GL = N // _NG
_NMIN = 4


def _knn_body(coords_ref, qt_ref, idx_ref, dist_ref):
    c = coords_ref[...]
    ksq = jnp.sum(c * c, axis=1, keepdims=True)
    qt = qt_ref[...]
    qsq = jnp.sum(qt * qt, axis=0, keepdims=True)
    st = (qsq + ksq) - 2.0 * jnp.dot(c, qt, preferred_element_type=jnp.float32)

    s3 = st.reshape(_NG, _GL, QB)
    io = lax.broadcasted_iota(jnp.int32, (_NG, _GL, QB), 1)
    go = lax.broadcasted_iota(jnp.int32, (_NG, QB), 0) * _GL
    cds, cis = [], []
    cur = s3
    for _ in range(_NMIN):
        m = jnp.min(cur, axis=1)
        am = jnp.min(jnp.where(cur == m[:, None, :], io, _GL), axis=1)
        cds.append(m)
        cis.append(am + go)
        cur = jnp.where(io == am[:, None, :], 1e30, cur)
    cd = jnp.concatenate(cds, axis=0)
    ci = jnp.concatenate(cis, axis=0)
    tv, ti = [], []
    for _ in range(KNB):
        m = jnp.min(cd, axis=0, keepdims=True)
        gi = jnp.min(jnp.where(cd == m, ci, N), axis=0, keepdims=True)
        tv.append(m)
        ti.append(gi)
        cd = jnp.where((cd == m) & (ci == gi), 1e30, cd)
    tau = tv[-1]
    cnt = jnp.sum((s3 <= tau[:, None, :]).astype(jnp.int32), axis=1)
    viol = jnp.any(cnt > _NMIN)
    tvc = jnp.concatenate(tv, axis=0)
    tic = jnp.concatenate(ti, axis=0)

    def _fallback(_):
        io2 = lax.broadcasted_iota(jnp.int32, (N, QB), 0)
        d = st
        fv, fi = [], []
        for _ in range(KNB):
            m = jnp.min(d, axis=0, keepdims=True)
            am = jnp.min(jnp.where(d == m, io2, N), axis=0, keepdims=True)
            fv.append(m)
            fi.append(am)
            d = jnp.where(io2 == am, 1e30, d)
        return jnp.concatenate(fv, axis=0), jnp.concatenate(fi, axis=0)

    tvf, tif = lax.cond(viol, _fallback, lambda _: (tvc, tic), 0)
    idx_ref[...] = tif
    dist_ref[...] = jnp.sqrt(jnp.maximum(tvf, 0.0))


def _knn(coords):
    ct = coords.T
    idxT, distT = pl.pallas_call(
        _knn_body,
        grid=(N // QB,),
        in_specs=[
            pl.BlockSpec((N, 3), lambda i: (0, 0)),
            pl.BlockSpec((3, QB), lambda i: (0, i)),
        ],
        out_specs=[
            pl.BlockSpec((KNB, QB), lambda i: (0, i)),
            pl.BlockSpec((KNB, QB), lambda i: (0, i)),
        ],
        out_shape=[
            jax.ShapeDtypeStruct((KNB, N), jnp.int32),
            jax.ShapeDtypeStruct((KNB, N), jnp.float32),
        ],
    )(coords, ct)
    return idxT, distT


_NW = 32
_CHUNK = 128
_PER_W = NK // _NW


def _sc_gather_t(table, idxf):
    mesh = plsc.VectorSubcoreMesh(core_axis_name="c", subcore_axis_name="s")

    @functools.partial(
        pl.kernel, mesh=mesh,
        out_type=jax.ShapeDtypeStruct((NK, 128), jnp.float32),
        scratch_types=[
            pltpu.VMEM((_CHUNK,), jnp.int32),
            pltpu.VMEM((_CHUNK, 128), jnp.float32),
            pltpu.SemaphoreType.DMA,
        ],
    )
    def k(table_hbm, idx_hbm, out_hbm, idx_c, rows, sem):
        wid = lax.axis_index("s") * 2 + lax.axis_index("c")
        base = wid * _PER_W
        nch = _PER_W // _CHUNK

        def body(c, _):
            off = pl.multiple_of(base + c * _CHUNK, _CHUNK)
            pltpu.sync_copy(idx_hbm.at[pl.ds(off, _CHUNK)], idx_c)
            pltpu.async_copy(table_hbm.at[idx_c], rows, sem).wait()
            pltpu.sync_copy(rows, out_hbm.at[pl.ds(off, _CHUNK)])
            return _

        lax.fori_loop(0, nch, body, 0)

    return k(table, idxf)


def _sc_gather_nf2(y1p, idxf, sc16, sh16):
    mesh = plsc.VectorSubcoreMesh(core_axis_name="c", subcore_axis_name="s")
    aff = jnp.concatenate([sc16[None], sh16[None]], axis=0)

    @functools.partial(
        pl.kernel, mesh=mesh,
        out_type=jax.ShapeDtypeStruct((N, 16), jnp.float32),
        scratch_types=[
            pltpu.VMEM((_CHUNK,), jnp.int32),
            pltpu.VMEM((_CHUNK, 128), jnp.float32),
            pltpu.VMEM((2, 16), jnp.float32),
            pltpu.VMEM((8, 16), jnp.float32),
            pltpu.SemaphoreType.DMA,
        ],
    )
    def k(y1_hbm, idx_hbm, aff_hbm, out_hbm, idx_c, rows, aff_v, acc_v, sem):
        wid = lax.axis_index("s") * 2 + lax.axis_index("c")
        base = wid * _PER_W
        pltpu.sync_copy(aff_hbm, aff_v)
        nch = _PER_W // _CHUNK

        def body(c, _):
            off = pl.multiple_of(base + c * _CHUNK, _CHUNK)
            pltpu.sync_copy(idx_hbm.at[pl.ds(off, _CHUNK)], idx_c)
            pltpu.async_copy(y1_hbm.at[idx_c], rows, sem).wait()
            sc = aff_v[0, :]
            sh = aff_v[1, :]
            for i in range(_CHUNK // KNB):
                acc = jnp.zeros((16,), jnp.float32)
                for j in range(KNB):
                    r = rows[i * KNB + j, 0:16]
                    acc = acc + jnp.maximum(r * sc + sh, 0.0)
                acc_v[i, :] = acc * (1.0 / KNB)
            out_rowbase = pl.multiple_of((base + c * _CHUNK) // KNB, 8)
            pltpu.sync_copy(acc_v, out_hbm.at[pl.ds(out_rowbase, 8)])
            return _

        lax.fori_loop(0, nch, body, 0)

    return k(y1p, idxf, aff)


def _p1_body(coords_ref, featT_ref, vw_ref, uw_ref, f1w_ref, b1_ref,
             wsT_ref, t_ref, u_ref, ssst_ref):
    i = pl.program_id(0)
    c = coords_ref[...]
    ft = featT_ref[...]
    v = jnp.dot(c, vw_ref[...], preferred_element_type=jnp.float32)
    f1p = jnp.dot(ft, f1w_ref[...], preferred_element_type=jnp.float32)
    f1p = f1p + b1_ref[0:1, :]
    f1 = jnp.where(f1p > 0, f1p, 0.2 * f1p)
    t_ref[...] = jnp.concatenate(
        [v, f1, jnp.zeros((v.shape[0], 80), jnp.float32)], axis=1)
    u = jnp.dot(c, uw_ref[...], preferred_element_type=jnp.float32)
    u_ref[...] = jnp.concatenate([u, jnp.zeros_like(f1)], axis=1)
    ss = jnp.dot(ft, wsT_ref[...], preferred_element_type=jnp.float32)

    @pl.when(i == 0)
    def _():
        ssst_ref[...] = jnp.zeros_like(ssst_ref)

    s1 = jnp.sum(ss, axis=0)
    s2 = jnp.sum(ss * ss, axis=0)
    pad = jnp.zeros((6, 64), jnp.float32)
    ssst_ref[...] += jnp.concatenate([s1[None], s2[None], pad], axis=0)


def _p1(coords, featT, vw, uw, f1w, b1row, wsT):
    PB = 2048
    return pl.pallas_call(
        _p1_body,
        grid=(N // PB,),
        in_specs=[
            pl.BlockSpec((PB, 3), lambda i: (i, 0)),
            pl.BlockSpec((PB, 8), lambda i: (i, 0)),
            pl.BlockSpec((3, 32), lambda i: (0, 0)),
            pl.BlockSpec((3, 32), lambda i: (0, 0)),
            pl.BlockSpec((8, 16), lambda i: (0, 0)),
            pl.BlockSpec((8, 16), lambda i: (0, 0)),
            pl.BlockSpec((8, 64), lambda i: (0, 0)),
        ],
        out_specs=[
            pl.BlockSpec((PB, 128), lambda i: (i, 0)),
            pl.BlockSpec((PB, 48), lambda i: (i, 0)),
            pl.BlockSpec((8, 64), lambda i: (0, 0)),
        ],
        out_shape=[
            jax.ShapeDtypeStruct((N, 128), jnp.float32),
            jax.ShapeDtypeStruct((N, 48), jnp.float32),
            jax.ShapeDtypeStruct((8, 64), jnp.float32),
        ],
    )(coords, featT, vw, uw, f1w, b1row, wsT)


def _passA_body(g_ref, u_ref, d_ref, wall_ref, tst_ref, nf1_ref):
    i = pl.program_id(0)
    t3 = (g_ref[...][:, :, 0:48] + u_ref[...][:, None, :]
          + d_ref[...][:, :, None] * wall_ref[0:1, :][None, :, :])

    @pl.when(i == 0)
    def _():
        tst_ref[...] = jnp.zeros_like(tst_ref)

    s1 = jnp.sum(t3, axis=(0, 1))
    s2 = jnp.sum(t3 * t3, axis=(0, 1))
    pad = jnp.zeros((6, 48), jnp.float32)
    tst_ref[...] += jnp.concatenate([s1[None], s2[None], pad], axis=0)
    nf1_ref[...] = jnp.mean(t3[:, :, 32:48], axis=1)


def _passA(g3, u48, dist, wall):
    return pl.pallas_call(
        _passA_body,
        grid=(N // AB,),
        in_specs=[
            pl.BlockSpec((AB, KNB, 128), lambda i: (i, 0, 0)),
            pl.BlockSpec((AB, 48), lambda i: (i, 0)),
            pl.BlockSpec((AB, KNB), lambda i: (i, 0)),
            pl.BlockSpec((8, 48), lambda i: (0, 0)),
        ],
        out_specs=[
            pl.BlockSpec((8, 48), lambda i: (0, 0)),
            pl.BlockSpec((AB, 16), lambda i: (i, 0)),
        ],
        out_shape=[
            jax.ShapeDtypeStruct((8, 48), jnp.float32),
            jax.ShapeDtypeStruct((N, 16), jnp.float32),
        ],
    )(g3, u48, dist, wall)


def _passB1_body(g_ref, u_ref, d_ref, wall_ref, aff_ref, nf1_ref, wp1_ref,
                 y1_ref, mh2_ref, yst_ref):
    i = pl.program_id(0)
    t3 = (g_ref[...][:, :, 0:48] + u_ref[...][:, None, :]
          + d_ref[...][:, :, None] * wall_ref[0:1, :][None, :, :])
    h3 = t3 * aff_ref[0:1, :][None, :, :] + aff_ref[1:2, :][None, :, :]
    h3 = jnp.maximum(h3, 0.0)
    mh = jnp.mean(h3, axis=1)
    pooled1 = jnp.concatenate([mh[:, 0:16], nf1_ref[...]], axis=1)
    y1 = jnp.dot(pooled1, wp1_ref[...], preferred_element_type=jnp.float32)
    y1_ref[...] = y1
    mh2_ref[...] = mh[:, 16:32]

    @pl.when(i == 0)
    def _():
        yst_ref[...] = jnp.zeros_like(yst_ref)

    s1 = jnp.sum(y1, axis=0)
    s2 = jnp.sum(y1 * y1, axis=0)
    pad = jnp.zeros((6, 16), jnp.float32)
    yst_ref[...] += jnp.concatenate([s1[None], s2[None], pad], axis=0)


def _passB1(g3, u48, dist, wall, aff48, nf1, wp1T):
    return pl.pallas_call(
        _passB1_body,
        grid=(N // AB,),
        in_specs=[
            pl.BlockSpec((AB, KNB, 128), lambda i: (i, 0, 0)),
            pl.BlockSpec((AB, 48), lambda i: (i, 0)),
            pl.BlockSpec((AB, KNB), lambda i: (i, 0)),
            pl.BlockSpec((8, 48), lambda i: (0, 0)),
            pl.BlockSpec((8, 48), lambda i: (0, 0)),
            pl.BlockSpec((AB, 16), lambda i: (i, 0)),
            pl.BlockSpec((32, 16), lambda i: (0, 0)),
        ],
        out_specs=[
            pl.BlockSpec((AB, 16), lambda i: (i, 0)),
            pl.BlockSpec((AB, 16), lambda i: (i, 0)),
            pl.BlockSpec((8, 16), lambda i: (0, 0)),
        ],
        out_shape=[
            jax.ShapeDtypeStruct((N, 16), jnp.float32),
            jax.ShapeDtypeStruct((N, 16), jnp.float32),
            jax.ShapeDtypeStruct((8, 16), jnp.float32),
        ],
    )(g3, u48, dist, wall, aff48, nf1, wp1T)


def _passB3_body(mh2_ref, nf2_ref, wp2_ref, y2_ref, yst_ref):
    i = pl.program_id(0)
    pooled2 = jnp.concatenate([mh2_ref[...], nf2_ref[...]], axis=1)
    y2 = jnp.dot(pooled2, wp2_ref[...], preferred_element_type=jnp.float32)
    y2_ref[...] = y2

    @pl.when(i == 0)
    def _():
        yst_ref[...] = jnp.zeros_like(yst_ref)

    s1 = jnp.sum(y2, axis=0)
    s2 = jnp.sum(y2 * y2, axis=0)
    pad = jnp.zeros((6, 32), jnp.float32)
    yst_ref[...] += jnp.concatenate([s1[None], s2[None], pad], axis=0)


def _passB3(mh2, nf2, wp2T):
    return pl.pallas_call(
        _passB3_body,
        grid=(N // CB,),
        in_specs=[
            pl.BlockSpec((CB, 16), lambda i: (i, 0)),
            pl.BlockSpec((CB, 16), lambda i: (i, 0)),
            pl.BlockSpec((32, 32), lambda i: (0, 0)),
        ],
        out_specs=[
            pl.BlockSpec((CB, 32), lambda i: (i, 0)),
            pl.BlockSpec((8, 32), lambda i: (0, 0)),
        ],
        out_shape=[
            jax.ShapeDtypeStruct((N, 32), jnp.float32),
            jax.ShapeDtypeStruct((8, 32), jnp.float32),
        ],
    )(mh2, nf2, wp2T)


def _passB4_body(y2_ref, f8_ref, aff2_ref, ws_ref, w2_ref, saff_ref, b2_ref,
                 out_ref):
    y2 = y2_ref[...]
    x2 = jnp.maximum(y2 * aff2_ref[0:1, :] + aff2_ref[1:2, :], 0.0)
    ss = jnp.dot(ws_ref[...], f8_ref[...], preferred_element_type=jnp.float32)
    ssn = ss * saff_ref[:, 0:1] + saff_ref[:, 1:2]
    o = lax.dot_general(w2_ref[...], x2, (((1,), (1,)), ((), ())),
                        preferred_element_type=jnp.float32)
    o = o + b2_ref[...] + ssn
    out_ref[...] = jnp.where(o > 0, o, 0.01 * o)


def _passB4(y2, feat8, aff2, ws, w2, saff, b2col):
    return pl.pallas_call(
        _passB4_body,
        grid=(N // CB,),
        in_specs=[
            pl.BlockSpec((CB, 32), lambda i: (i, 0)),
            pl.BlockSpec((8, CB), lambda i: (0, i)),
            pl.BlockSpec((8, 32), lambda i: (0, 0)),
            pl.BlockSpec((64, 8), lambda i: (0, 0)),
            pl.BlockSpec((64, 32), lambda i: (0, 0)),
            pl.BlockSpec((64, 8), lambda i: (0, 0)),
            pl.BlockSpec((64, 1), lambda i: (0, 0)),
        ],
        out_specs=pl.BlockSpec((64, CB), lambda i: (0, i)),
        out_shape=jax.ShapeDtypeStruct((64, N), jnp.float32),
    )(y2, feat8, aff2, ws, w2, saff, b2col)


def _pad8(row):
    return jnp.concatenate([row[None, :], jnp.zeros((7, row.shape[0]), row.dtype)], 0)


def _stats_to_aff(s1, s2, count, gw, gb, group_size):
    C = s1.shape[0]
    g1 = s1.reshape(C // group_size, group_size).sum(1)
    g2 = s2.reshape(C // group_size, group_size).sum(1)
    m = g1 / (count * group_size)
    var = g2 / (count * group_size) - m * m
    inv = 1.0 / jnp.sqrt(var + EPS)
    m = jnp.repeat(m, group_size)
    inv = jnp.repeat(inv, group_size)
    scale = gw * inv
    shift = gb - m * scale
    return scale, shift


def kernel(coords, features, W1, b1, Wl1, bl1, gl1w, gl1b, Wp1, bp1, gp1w, gp1b, Wl2, bl2, gl2w, gl2b, Wp2, bp2, gp2w, gp2b, W2, b2, Ws, bs, gsw, gsb):
    c0 = coords[0]
    feat8 = features[0, :, :, 0]
    featT = feat8.T

    idxT, distT = _knn(c0)
    idx = idxT.T
    dist = distT.T
    idxf = idx.reshape(-1)

    A1 = Wl1[:, 0:3] + Wl1[:, 6:9]
    B1m = Wl1[:, 3:6] - Wl1[:, 6:9]
    w1 = Wl1[:, 9]
    A2 = Wl2[:, 0:3] + Wl2[:, 6:9]
    B2m = Wl2[:, 3:6] - Wl2[:, 6:9]
    w2 = Wl2[:, 9]
    vw = jnp.concatenate([B1m.T, B2m.T], axis=1)
    uw = jnp.concatenate([A1.T, A2.T], axis=1)
    wall = _pad8(jnp.concatenate([w1, w2, jnp.zeros(16, jnp.float32)]))

    T, U48, ssst = _p1(c0, featT, vw, uw, W1.T, _pad8(b1), Ws.T)

    g3 = _sc_gather_t(T, idxf).reshape(N, KNB, 128)

    tst, nf1 = _passA(g3, U48, dist, wall)

    gl12w = jnp.concatenate([gl1w, gl2w, jnp.zeros(16, jnp.float32)])
    gl12b = jnp.concatenate([gl1b, gl2b, jnp.zeros(16, jnp.float32)])
    sc48, sh48 = _stats_to_aff(tst[0], tst[1], float(NK), gl12w, gl12b, 1)
    aff48 = jnp.concatenate([sc48[None], sh48[None],
                             jnp.zeros((6, 48), jnp.float32)], 0)

    y1, mh2, y1st = _passB1(g3, U48, dist, wall, aff48, nf1, Wp1.T)

    sc1, shf1 = _stats_to_aff(y1st[0], y1st[1], float(N), gp1w, gp1b, 1)
    y1p = jnp.concatenate([y1, jnp.zeros((N, 112), jnp.float32)], axis=1)
    nf2 = _sc_gather_nf2(y1p, idxf, sc1, shf1)

    y2, y2st = _passB3(mh2, nf2, Wp2.T)

    sc2, shf2 = _stats_to_aff(y2st[0], y2st[1], float(N), gp2w, gp2b, 2)
    aff2 = jnp.concatenate([sc2[None], shf2[None],
                            jnp.zeros((6, 32), jnp.float32)], 0)
    scs, shs = _stats_to_aff(ssst[0], ssst[1], float(N), gsw, gsb, 4)
    saff = jnp.concatenate([scs[:, None], shs[:, None],
                            jnp.zeros((64, 6), jnp.float32)], 1)

    out = _passB4(y2, feat8, aff2, Ws, W2, saff, b2[:, None])
    return out[None, :, :, None]


# speedup vs baseline: 7.1047x; 1.0007x over previous
import functools

import jax
import jax.numpy as jnp
from jax import lax
from jax.experimental import pallas as pl
from jax.experimental.pallas import tpu as pltpu
from jax.experimental.pallas import tpu_sc as plsc

N = 16384
KNB = 16
NK = N * KNB
EPS = 1e-6
QB = 128
AB = 256
CB = 512


_NG = 128
_GL = N // _NG
_NMIN = 4


def _knn_body(coords_ref, qt_ref, idx_ref, dist_ref):
    c = coords_ref[...]
    ksq = jnp.sum(c * c, axis=1, keepdims=True)
    qt = qt_ref[...]
    qsq = jnp.sum(qt * qt, axis=0, keepdims=True)
    st = (qsq + ksq) - 2.0 * jnp.dot(c, qt, preferred_element_type=jnp.float32)

    s3 = st.reshape(_NG, _GL, QB)
    io = lax.broadcasted_iota(jnp.int32, (_NG, _GL, QB), 1)
    go = lax.broadcasted_iota(jnp.int32, (_NG, QB), 0) * _GL
    cds, cis = [], []
    cur = s3
    for _ in range(_NMIN):
        m = jnp.min(cur, axis=1)
        am = jnp.min(jnp.where(cur == m[:, None, :], io, _GL), axis=1)
        cds.append(m)
        cis.append(am + go)
        cur = jnp.where(io == am[:, None, :], 1e30, cur)
    cd = jnp.concatenate(cds, axis=0)
    ci = jnp.concatenate(cis, axis=0)
    tv, ti = [], []
    for _ in range(KNB):
        m = jnp.min(cd, axis=0, keepdims=True)
        gi = jnp.min(jnp.where(cd == m, ci, N), axis=0, keepdims=True)
        tv.append(m)
        ti.append(gi)
        cd = jnp.where((cd == m) & (ci == gi), 1e30, cd)
    tau = tv[-1]
    cnt = jnp.sum((s3 <= tau[:, None, :]).astype(jnp.int32), axis=1)
    viol = jnp.any(cnt > _NMIN)
    tvc = jnp.concatenate(tv, axis=0)
    tic = jnp.concatenate(ti, axis=0)
    idx_ref[...] = tic
    dist_ref[...] = jnp.sqrt(jnp.maximum(tvc, 0.0))

    @pl.when(viol)
    def _fallback():
        io2 = lax.broadcasted_iota(jnp.int32, (N, QB), 0)
        d = st
        fv, fi = [], []
        for _ in range(KNB):
            m = jnp.min(d, axis=0, keepdims=True)
            am = jnp.min(jnp.where(d == m, io2, N), axis=0, keepdims=True)
            fv.append(m)
            fi.append(am)
            d = jnp.where(io2 == am, 1e30, d)
        idx_ref[...] = jnp.concatenate(fi, axis=0)
        dist_ref[...] = jnp.sqrt(
            jnp.maximum(jnp.concatenate(fv, axis=0), 0.0))


def _knn(coords):
    ct = coords.T
    idxT, distT = pl.pallas_call(
        _knn_body,
        grid=(N // QB,),
        in_specs=[
            pl.BlockSpec((N, 3), lambda i: (0, 0)),
            pl.BlockSpec((3, QB), lambda i: (0, i)),
        ],
        out_specs=[
            pl.BlockSpec((KNB, QB), lambda i: (0, i)),
            pl.BlockSpec((KNB, QB), lambda i: (0, i)),
        ],
        out_shape=[
            jax.ShapeDtypeStruct((KNB, N), jnp.int32),
            jax.ShapeDtypeStruct((KNB, N), jnp.float32),
        ],
    )(coords, ct)
    return idxT, distT


_NW = 32
_CHUNK = 128
_PER_W = NK // _NW


def _sc_gather_t(table, idxf):
    mesh = plsc.VectorSubcoreMesh(core_axis_name="c", subcore_axis_name="s")

    @functools.partial(
        pl.kernel, mesh=mesh,
        out_type=jax.ShapeDtypeStruct((NK, 128), jnp.float32),
        scratch_types=[
            pltpu.VMEM((_CHUNK,), jnp.int32),
            pltpu.VMEM((_CHUNK, 128), jnp.float32),
            pltpu.SemaphoreType.DMA,
        ],
    )
    def k(table_hbm, idx_hbm, out_hbm, idx_c, rows, sem):
        wid = lax.axis_index("s") * 2 + lax.axis_index("c")
        base = wid * _PER_W
        nch = _PER_W // _CHUNK

        def body(c, _):
            off = pl.multiple_of(base + c * _CHUNK, _CHUNK)
            pltpu.sync_copy(idx_hbm.at[pl.ds(off, _CHUNK)], idx_c)
            pltpu.async_copy(table_hbm.at[idx_c], rows, sem).wait()
            pltpu.sync_copy(rows, out_hbm.at[pl.ds(off, _CHUNK)])
            return _

        lax.fori_loop(0, nch, body, 0)

    return k(table, idxf)


def _sc_gather_nf2(y1p, idxf, sc16, sh16):
    mesh = plsc.VectorSubcoreMesh(core_axis_name="c", subcore_axis_name="s")
    aff = jnp.concatenate([sc16[None], sh16[None]], axis=0)

    @functools.partial(
        pl.kernel, mesh=mesh,
        out_type=jax.ShapeDtypeStruct((N, 16), jnp.float32),
        scratch_types=[
            pltpu.VMEM((_CHUNK,), jnp.int32),
            pltpu.VMEM((_CHUNK, 128), jnp.float32),
            pltpu.VMEM((2, 16), jnp.float32),
            pltpu.VMEM((8, 16), jnp.float32),
            pltpu.SemaphoreType.DMA,
        ],
    )
    def k(y1_hbm, idx_hbm, aff_hbm, out_hbm, idx_c, rows, aff_v, acc_v, sem):
        wid = lax.axis_index("s") * 2 + lax.axis_index("c")
        base = wid * _PER_W
        pltpu.sync_copy(aff_hbm, aff_v)
        nch = _PER_W // _CHUNK

        def body(c, _):
            off = pl.multiple_of(base + c * _CHUNK, _CHUNK)
            pltpu.sync_copy(idx_hbm.at[pl.ds(off, _CHUNK)], idx_c)
            pltpu.async_copy(y1_hbm.at[idx_c], rows, sem).wait()
            sc = aff_v[0, :]
            sh = aff_v[1, :]
            for i in range(_CHUNK // KNB):
                acc = jnp.zeros((16,), jnp.float32)
                for j in range(KNB):
                    r = rows[i * KNB + j, 0:16]
                    acc = acc + jnp.maximum(r * sc + sh, 0.0)
                acc_v[i, :] = acc * (1.0 / KNB)
            out_rowbase = pl.multiple_of((base + c * _CHUNK) // KNB, 8)
            pltpu.sync_copy(acc_v, out_hbm.at[pl.ds(out_rowbase, 8)])
            return _

        lax.fori_loop(0, nch, body, 0)

    return k(y1p, idxf, aff)


def _p1_body(coords_ref, featT_ref, vw_ref, uw_ref, f1w_ref, b1_ref,
             wsT_ref, t_ref, u_ref, ssst_ref):
    i = pl.program_id(0)
    c = coords_ref[...]
    ft = featT_ref[...]
    v = jnp.dot(c, vw_ref[...], preferred_element_type=jnp.float32)
    f1p = jnp.dot(ft, f1w_ref[...], preferred_element_type=jnp.float32)
    f1p = f1p + b1_ref[0:1, :]
    f1 = jnp.where(f1p > 0, f1p, 0.2 * f1p)
    t_ref[...] = jnp.concatenate(
        [v, f1, jnp.zeros((v.shape[0], 80), jnp.float32)], axis=1)
    u = jnp.dot(c, uw_ref[...], preferred_element_type=jnp.float32)
    u_ref[...] = jnp.concatenate([u, jnp.zeros_like(f1)], axis=1)
    ss = jnp.dot(ft, wsT_ref[...], preferred_element_type=jnp.float32)

    @pl.when(i == 0)
    def _():
        ssst_ref[...] = jnp.zeros_like(ssst_ref)

    s1 = jnp.sum(ss, axis=0)
    s2 = jnp.sum(ss * ss, axis=0)
    pad = jnp.zeros((6, 64), jnp.float32)
    ssst_ref[...] += jnp.concatenate([s1[None], s2[None], pad], axis=0)


def _p1(coords, featT, vw, uw, f1w, b1row, wsT):
    PB = 2048
    return pl.pallas_call(
        _p1_body,
        grid=(N // PB,),
        in_specs=[
            pl.BlockSpec((PB, 3), lambda i: (i, 0)),
            pl.BlockSpec((PB, 8), lambda i: (i, 0)),
            pl.BlockSpec((3, 32), lambda i: (0, 0)),
            pl.BlockSpec((3, 32), lambda i: (0, 0)),
            pl.BlockSpec((8, 16), lambda i: (0, 0)),
            pl.BlockSpec((8, 16), lambda i: (0, 0)),
            pl.BlockSpec((8, 64), lambda i: (0, 0)),
        ],
        out_specs=[
            pl.BlockSpec((PB, 128), lambda i: (i, 0)),
            pl.BlockSpec((PB, 48), lambda i: (i, 0)),
            pl.BlockSpec((8, 64), lambda i: (0, 0)),
        ],
        out_shape=[
            jax.ShapeDtypeStruct((N, 128), jnp.float32),
            jax.ShapeDtypeStruct((N, 48), jnp.float32),
            jax.ShapeDtypeStruct((8, 64), jnp.float32),
        ],
    )(coords, featT, vw, uw, f1w, b1row, wsT)


def _passA_body(g_ref, u_ref, d_ref, wall_ref, tst_ref, nf1_ref):
    i = pl.program_id(0)
    t3 = (g_ref[...][:, :, 0:48] + u_ref[...][:, None, :]
          + d_ref[...][:, :, None] * wall_ref[0:1, :][None, :, :])

    @pl.when(i == 0)
    def _():
        tst_ref[...] = jnp.zeros_like(tst_ref)

    s1 = jnp.sum(t3, axis=(0, 1))
    s2 = jnp.sum(t3 * t3, axis=(0, 1))
    pad = jnp.zeros((6, 48), jnp.float32)
    tst_ref[...] += jnp.concatenate([s1[None], s2[None], pad], axis=0)
    nf1_ref[...] = jnp.mean(t3[:, :, 32:48], axis=1)


def _passA(g3, u48, dist, wall):
    return pl.pallas_call(
        _passA_body,
        grid=(N // AB,),
        in_specs=[
            pl.BlockSpec((AB, KNB, 128), lambda i: (i, 0, 0)),
            pl.BlockSpec((AB, 48), lambda i: (i, 0)),
            pl.BlockSpec((AB, KNB), lambda i: (i, 0)),
            pl.BlockSpec((8, 48), lambda i: (0, 0)),
        ],
        out_specs=[
            pl.BlockSpec((8, 48), lambda i: (0, 0)),
            pl.BlockSpec((AB, 16), lambda i: (i, 0)),
        ],
        out_shape=[
            jax.ShapeDtypeStruct((8, 48), jnp.float32),
            jax.ShapeDtypeStruct((N, 16), jnp.float32),
        ],
    )(g3, u48, dist, wall)


def _passB1_body(g_ref, u_ref, d_ref, wall_ref, aff_ref, nf1_ref, wp1_ref,
                 y1_ref, mh2_ref, yst_ref):
    i = pl.program_id(0)
    t3 = (g_ref[...][:, :, 0:48] + u_ref[...][:, None, :]
          + d_ref[...][:, :, None] * wall_ref[0:1, :][None, :, :])
    h3 = t3 * aff_ref[0:1, :][None, :, :] + aff_ref[1:2, :][None, :, :]
    h3 = jnp.maximum(h3, 0.0)
    mh = jnp.mean(h3, axis=1)
    pooled1 = jnp.concatenate([mh[:, 0:16], nf1_ref[...]], axis=1)
    y1 = jnp.dot(pooled1, wp1_ref[...], preferred_element_type=jnp.float32)
    y1_ref[...] = y1
    mh2_ref[...] = mh[:, 16:32]

    @pl.when(i == 0)
    def _():
        yst_ref[...] = jnp.zeros_like(yst_ref)

    s1 = jnp.sum(y1, axis=0)
    s2 = jnp.sum(y1 * y1, axis=0)
    pad = jnp.zeros((6, 16), jnp.float32)
    yst_ref[...] += jnp.concatenate([s1[None], s2[None], pad], axis=0)


def _passB1(g3, u48, dist, wall, aff48, nf1, wp1T):
    return pl.pallas_call(
        _passB1_body,
        grid=(N // AB,),
        in_specs=[
            pl.BlockSpec((AB, KNB, 128), lambda i: (i, 0, 0)),
            pl.BlockSpec((AB, 48), lambda i: (i, 0)),
            pl.BlockSpec((AB, KNB), lambda i: (i, 0)),
            pl.BlockSpec((8, 48), lambda i: (0, 0)),
            pl.BlockSpec((8, 48), lambda i: (0, 0)),
            pl.BlockSpec((AB, 16), lambda i: (i, 0)),
            pl.BlockSpec((32, 16), lambda i: (0, 0)),
        ],
        out_specs=[
            pl.BlockSpec((AB, 16), lambda i: (i, 0)),
            pl.BlockSpec((AB, 16), lambda i: (i, 0)),
            pl.BlockSpec((8, 16), lambda i: (0, 0)),
        ],
        out_shape=[
            jax.ShapeDtypeStruct((N, 16), jnp.float32),
            jax.ShapeDtypeStruct((N, 16), jnp.float32),
            jax.ShapeDtypeStruct((8, 16), jnp.float32),
        ],
    )(g3, u48, dist, wall, aff48, nf1, wp1T)


def _passB3_body(mh2_ref, nf2_ref, wp2_ref, y2_ref, yst_ref):
    i = pl.program_id(0)
    pooled2 = jnp.concatenate([mh2_ref[...], nf2_ref[...]], axis=1)
    y2 = jnp.dot(pooled2, wp2_ref[...], preferred_element_type=jnp.float32)
    y2_ref[...] = y2

    @pl.when(i == 0)
    def _():
        yst_ref[...] = jnp.zeros_like(yst_ref)

    s1 = jnp.sum(y2, axis=0)
    s2 = jnp.sum(y2 * y2, axis=0)
    pad = jnp.zeros((6, 32), jnp.float32)
    yst_ref[...] += jnp.concatenate([s1[None], s2[None], pad], axis=0)


def _passB3(mh2, nf2, wp2T):
    return pl.pallas_call(
        _passB3_body,
        grid=(N // CB,),
        in_specs=[
            pl.BlockSpec((CB, 16), lambda i: (i, 0)),
            pl.BlockSpec((CB, 16), lambda i: (i, 0)),
            pl.BlockSpec((32, 32), lambda i: (0, 0)),
        ],
        out_specs=[
            pl.BlockSpec((CB, 32), lambda i: (i, 0)),
            pl.BlockSpec((8, 32), lambda i: (0, 0)),
        ],
        out_shape=[
            jax.ShapeDtypeStruct((N, 32), jnp.float32),
            jax.ShapeDtypeStruct((8, 32), jnp.float32),
        ],
    )(mh2, nf2, wp2T)


def _passB4_body(y2_ref, f8_ref, aff2_ref, ws_ref, w2_ref, saff_ref, b2_ref,
                 out_ref):
    y2 = y2_ref[...]
    x2 = jnp.maximum(y2 * aff2_ref[0:1, :] + aff2_ref[1:2, :], 0.0)
    ss = jnp.dot(ws_ref[...], f8_ref[...], preferred_element_type=jnp.float32)
    ssn = ss * saff_ref[:, 0:1] + saff_ref[:, 1:2]
    o = lax.dot_general(w2_ref[...], x2, (((1,), (1,)), ((), ())),
                        preferred_element_type=jnp.float32)
    o = o + b2_ref[...] + ssn
    out_ref[...] = jnp.where(o > 0, o, 0.01 * o)


def _passB4(y2, feat8, aff2, ws, w2, saff, b2col):
    return pl.pallas_call(
        _passB4_body,
        grid=(N // CB,),
        in_specs=[
            pl.BlockSpec((CB, 32), lambda i: (i, 0)),
            pl.BlockSpec((8, CB), lambda i: (0, i)),
            pl.BlockSpec((8, 32), lambda i: (0, 0)),
            pl.BlockSpec((64, 8), lambda i: (0, 0)),
            pl.BlockSpec((64, 32), lambda i: (0, 0)),
            pl.BlockSpec((64, 8), lambda i: (0, 0)),
            pl.BlockSpec((64, 1), lambda i: (0, 0)),
        ],
        out_specs=pl.BlockSpec((64, CB), lambda i: (0, i)),
        out_shape=jax.ShapeDtypeStruct((64, N), jnp.float32),
    )(y2, feat8, aff2, ws, w2, saff, b2col)


def _pad8(row):
    return jnp.concatenate([row[None, :], jnp.zeros((7, row.shape[0]), row.dtype)], 0)


def _stats_to_aff(s1, s2, count, gw, gb, group_size):
    C = s1.shape[0]
    g1 = s1.reshape(C // group_size, group_size).sum(1)
    g2 = s2.reshape(C // group_size, group_size).sum(1)
    m = g1 / (count * group_size)
    var = g2 / (count * group_size) - m * m
    inv = 1.0 / jnp.sqrt(var + EPS)
    m = jnp.repeat(m, group_size)
    inv = jnp.repeat(inv, group_size)
    scale = gw * inv
    shift = gb - m * scale
    return scale, shift


def kernel(coords, features, W1, b1, Wl1, bl1, gl1w, gl1b, Wp1, bp1, gp1w, gp1b, Wl2, bl2, gl2w, gl2b, Wp2, bp2, gp2w, gp2b, W2, b2, Ws, bs, gsw, gsb):
    c0 = coords[0]
    feat8 = features[0, :, :, 0]
    featT = feat8.T

    idxT, distT = _knn(c0)
    idx = idxT.T
    dist = distT.T
    idxf = idx.reshape(-1)

    A1 = Wl1[:, 0:3] + Wl1[:, 6:9]
    B1m = Wl1[:, 3:6] - Wl1[:, 6:9]
    w1 = Wl1[:, 9]
    A2 = Wl2[:, 0:3] + Wl2[:, 6:9]
    B2m = Wl2[:, 3:6] - Wl2[:, 6:9]
    w2 = Wl2[:, 9]
    vw = jnp.concatenate([B1m.T, B2m.T], axis=1)
    uw = jnp.concatenate([A1.T, A2.T], axis=1)
    wall = _pad8(jnp.concatenate([w1, w2, jnp.zeros(16, jnp.float32)]))

    T, U48, ssst = _p1(c0, featT, vw, uw, W1.T, _pad8(b1), Ws.T)

    g3 = _sc_gather_t(T, idxf).reshape(N, KNB, 128)

    tst, nf1 = _passA(g3, U48, dist, wall)

    gl12w = jnp.concatenate([gl1w, gl2w, jnp.zeros(16, jnp.float32)])
    gl12b = jnp.concatenate([gl1b, gl2b, jnp.zeros(16, jnp.float32)])
    sc48, sh48 = _stats_to_aff(tst[0], tst[1], float(NK), gl12w, gl12b, 1)
    aff48 = jnp.concatenate([sc48[None], sh48[None],
                             jnp.zeros((6, 48), jnp.float32)], 0)

    y1, mh2, y1st = _passB1(g3, U48, dist, wall, aff48, nf1, Wp1.T)

    sc1, shf1 = _stats_to_aff(y1st[0], y1st[1], float(N), gp1w, gp1b, 1)
    y1p = jnp.concatenate([y1, jnp.zeros((N, 112), jnp.float32)], axis=1)
    nf2 = _sc_gather_nf2(y1p, idxf, sc1, shf1)

    y2, y2st = _passB3(mh2, nf2, Wp2.T)

    sc2, shf2 = _stats_to_aff(y2st[0], y2st[1], float(N), gp2w, gp2b, 2)
    aff2 = jnp.concatenate([sc2[None], shf2[None],
                            jnp.zeros((6, 32), jnp.float32)], 0)
    scs, shs = _stats_to_aff(ssst[0], ssst[1], float(N), gsw, gsb, 4)
    saff = jnp.concatenate([scs[:, None], shs[:, None],
                            jnp.zeros((64, 6), jnp.float32)], 1)

    out = _passB4(y2, feat8, aff2, Ws, W2, saff, b2[:, None])
    return out[None, :, :, None]


# speedup vs baseline: 16.0688x; 2.2617x over previous
import functools

import jax
import jax.numpy as jnp
from jax import lax
from jax.experimental import pallas as pl
from jax.experimental.pallas import tpu as pltpu
from jax.experimental.pallas import tpu_sc as plsc

N = 16384
KNB = 16
NK = N * KNB
EPS = 1e-6
QB = 128
AB = 256
CB = 512


_NG = 128
_GL = N // _NG
_NMIN = 4


def _knn_body(coords_ref, qt_ref, idx_ref, dist_ref):
    c = coords_ref[...]
    ksq = jnp.sum(c * c, axis=1, keepdims=True)
    qt = qt_ref[...]
    qsq = jnp.sum(qt * qt, axis=0, keepdims=True)
    st = (qsq + ksq) - 2.0 * jnp.dot(c, qt, preferred_element_type=jnp.float32)

    s3 = st.reshape(_NG, _GL, QB)
    ig = (lax.broadcasted_iota(jnp.int32, (_NG, _GL, QB), 0) * _GL
          + lax.broadcasted_iota(jnp.int32, (_NG, _GL, QB), 1))

    def _fex(av, ai, bv, bi):
        c = av <= bv
        return (jnp.where(c, av, bv), jnp.where(c, ai, bi),
                jnp.where(c, bv, av), jnp.where(c, bi, ai))

    def _hex(av, ai, bv, bi):
        c = av <= bv
        return jnp.where(c, av, bv), jnp.where(c, ai, bi)

    sv = [s3[:, 8 * t:8 * (t + 1), :] for t in range(16)]
    si = [ig[:, 8 * t:8 * (t + 1), :] for t in range(16)]
    p2 = []
    for t in range(8):
        lo, loi, hi, hii = _fex(sv[2 * t], si[2 * t], sv[2 * t + 1], si[2 * t + 1])
        p2.append([(lo, loi), (hi, hii)])
    p4 = []
    for t in range(4):
        a, b = p2[2 * t], p2[2 * t + 1]
        l0, li0, h0, hi0 = _fex(a[0][0], a[0][1], b[1][0], b[1][1])
        l1, li1, h1, hi1 = _fex(a[1][0], a[1][1], b[0][0], b[0][1])
        s0, s0i, s1, s1i = _fex(l0, li0, l1, li1)
        s2, s2i, s3b, s3i = _fex(h0, hi0, h1, hi1)
        p4.append([(s0, s0i), (s1, s1i), (s2, s2i), (s3b, s3i)])
    def _merge44(a, b, sort_out):
        ls = [_hex(a[i][0], a[i][1], b[3 - i][0], b[3 - i][1]) for i in range(4)]
        if not sort_out:
            return ls
        r0 = _fex(ls[0][0], ls[0][1], ls[2][0], ls[2][1])
        r1 = _fex(ls[1][0], ls[1][1], ls[3][0], ls[3][1])
        q0 = _fex(r0[0], r0[1], r1[0], r1[1])
        q1 = _fex(r0[2], r0[3], r1[2], r1[3])
        return [(q0[0], q0[1]), (q0[2], q0[3]), (q1[0], q1[1]), (q1[2], q1[3])]

    m0 = _merge44(p4[0], p4[1], True)
    m1 = _merge44(p4[2], p4[3], True)
    fin = _merge44(m0, m1, False)
    cv = jnp.concatenate([f[0] for f in fin], axis=1)
    cidx = jnp.concatenate([f[1] for f in fin], axis=1)

    cds, cis = [], []
    cur = cv
    for _ in range(_NMIN):
        m = jnp.min(cur, axis=1)
        gi = jnp.min(jnp.where(cur == m[:, None, :], cidx, N), axis=1)
        cds.append(m)
        cis.append(gi)
        cur = jnp.where((cur == m[:, None, :]) & (cidx == gi[:, None, :]),
                        1e30, cur)
    cd = jnp.concatenate(cds, axis=0)
    ci = jnp.concatenate(cis, axis=0)
    tv, ti = [], []
    for _ in range(KNB):
        m = jnp.min(cd, axis=0, keepdims=True)
        gi = jnp.min(jnp.where(cd == m, ci, N), axis=0, keepdims=True)
        tv.append(m)
        ti.append(gi)
        cd = jnp.where((cd == m) & (ci == gi), 1e30, cd)
    tau = tv[-1]
    cnt = jnp.sum((s3 <= tau[:, None, :]).astype(jnp.int32), axis=1)
    viol = jnp.any(cnt > _NMIN)
    tvc = jnp.concatenate(tv, axis=0)
    tic = jnp.concatenate(ti, axis=0)
    idx_ref[...] = tic
    dist_ref[...] = jnp.sqrt(jnp.maximum(tvc, 0.0))

    @pl.when(viol)
    def _fallback():
        io2 = lax.broadcasted_iota(jnp.int32, (N, QB), 0)
        d = st
        fv, fi = [], []
        for _ in range(KNB):
            m = jnp.min(d, axis=0, keepdims=True)
            am = jnp.min(jnp.where(d == m, io2, N), axis=0, keepdims=True)
            fv.append(m)
            fi.append(am)
            d = jnp.where(io2 == am, 1e30, d)
        idx_ref[...] = jnp.concatenate(fi, axis=0)
        dist_ref[...] = jnp.sqrt(
            jnp.maximum(jnp.concatenate(fv, axis=0), 0.0))


def _knn(coords):
    ct = coords.T
    idxT, distT = pl.pallas_call(
        _knn_body,
        grid=(N // QB,),
        in_specs=[
            pl.BlockSpec((N, 3), lambda i: (0, 0)),
            pl.BlockSpec((3, QB), lambda i: (0, i)),
        ],
        out_specs=[
            pl.BlockSpec((KNB, QB), lambda i: (0, i)),
            pl.BlockSpec((KNB, QB), lambda i: (0, i)),
        ],
        out_shape=[
            jax.ShapeDtypeStruct((KNB, N), jnp.int32),
            jax.ShapeDtypeStruct((KNB, N), jnp.float32),
        ],
    )(coords, ct)
    return idxT, distT


_NW = 32
_CHUNK = 128
_PER_W = NK // _NW


def _sc_gather_t(table, idxf):
    mesh = plsc.VectorSubcoreMesh(core_axis_name="c", subcore_axis_name="s")

    @functools.partial(
        pl.kernel, mesh=mesh,
        out_type=jax.ShapeDtypeStruct((NK, 128), jnp.float32),
        scratch_types=[
            pltpu.VMEM((_CHUNK,), jnp.int32),
            pltpu.VMEM((_CHUNK, 128), jnp.float32),
            pltpu.SemaphoreType.DMA,
        ],
    )
    def k(table_hbm, idx_hbm, out_hbm, idx_c, rows, sem):
        wid = lax.axis_index("s") * 2 + lax.axis_index("c")
        base = wid * _PER_W
        nch = _PER_W // _CHUNK

        def body(c, _):
            off = pl.multiple_of(base + c * _CHUNK, _CHUNK)
            pltpu.sync_copy(idx_hbm.at[pl.ds(off, _CHUNK)], idx_c)
            pltpu.async_copy(table_hbm.at[idx_c], rows, sem).wait()
            pltpu.sync_copy(rows, out_hbm.at[pl.ds(off, _CHUNK)])
            return _

        lax.fori_loop(0, nch, body, 0)

    return k(table, idxf)


def _sc_gather_nf2(y1p, idxf, sc16, sh16):
    mesh = plsc.VectorSubcoreMesh(core_axis_name="c", subcore_axis_name="s")
    aff = jnp.concatenate([sc16[None], sh16[None]], axis=0)

    @functools.partial(
        pl.kernel, mesh=mesh,
        out_type=jax.ShapeDtypeStruct((N, 16), jnp.float32),
        scratch_types=[
            pltpu.VMEM((_CHUNK,), jnp.int32),
            pltpu.VMEM((_CHUNK, 128), jnp.float32),
            pltpu.VMEM((2, 16), jnp.float32),
            pltpu.VMEM((8, 16), jnp.float32),
            pltpu.SemaphoreType.DMA,
        ],
    )
    def k(y1_hbm, idx_hbm, aff_hbm, out_hbm, idx_c, rows, aff_v, acc_v, sem):
        wid = lax.axis_index("s") * 2 + lax.axis_index("c")
        base = wid * _PER_W
        pltpu.sync_copy(aff_hbm, aff_v)
        nch = _PER_W // _CHUNK

        def body(c, _):
            off = pl.multiple_of(base + c * _CHUNK, _CHUNK)
            pltpu.sync_copy(idx_hbm.at[pl.ds(off, _CHUNK)], idx_c)
            pltpu.async_copy(y1_hbm.at[idx_c], rows, sem).wait()
            sc = aff_v[0, :]
            sh = aff_v[1, :]
            for i in range(_CHUNK // KNB):
                acc = jnp.zeros((16,), jnp.float32)
                for j in range(KNB):
                    r = rows[i * KNB + j, 0:16]
                    acc = acc + jnp.maximum(r * sc + sh, 0.0)
                acc_v[i, :] = acc * (1.0 / KNB)
            out_rowbase = pl.multiple_of((base + c * _CHUNK) // KNB, 8)
            pltpu.sync_copy(acc_v, out_hbm.at[pl.ds(out_rowbase, 8)])
            return _

        lax.fori_loop(0, nch, body, 0)

    return k(y1p, idxf, aff)


def _p1_body(coords_ref, featT_ref, vw_ref, uw_ref, f1w_ref, b1_ref,
             wsT_ref, t_ref, u_ref, ssst_ref):
    i = pl.program_id(0)
    c = coords_ref[...]
    ft = featT_ref[...]
    v = jnp.dot(c, vw_ref[...], preferred_element_type=jnp.float32)
    f1p = jnp.dot(ft, f1w_ref[...], preferred_element_type=jnp.float32)
    f1p = f1p + b1_ref[0:1, :]
    f1 = jnp.where(f1p > 0, f1p, 0.2 * f1p)
    t_ref[...] = jnp.concatenate(
        [v, f1, jnp.zeros((v.shape[0], 80), jnp.float32)], axis=1)
    u = jnp.dot(c, uw_ref[...], preferred_element_type=jnp.float32)
    u_ref[...] = jnp.concatenate([u, jnp.zeros_like(f1)], axis=1)
    ss = jnp.dot(ft, wsT_ref[...], preferred_element_type=jnp.float32)

    @pl.when(i == 0)
    def _():
        ssst_ref[...] = jnp.zeros_like(ssst_ref)

    s1 = jnp.sum(ss, axis=0)
    s2 = jnp.sum(ss * ss, axis=0)
    pad = jnp.zeros((6, 64), jnp.float32)
    ssst_ref[...] += jnp.concatenate([s1[None], s2[None], pad], axis=0)


def _p1(coords, featT, vw, uw, f1w, b1row, wsT):
    PB = 2048
    return pl.pallas_call(
        _p1_body,
        grid=(N // PB,),
        in_specs=[
            pl.BlockSpec((PB, 3), lambda i: (i, 0)),
            pl.BlockSpec((PB, 8), lambda i: (i, 0)),
            pl.BlockSpec((3, 32), lambda i: (0, 0)),
            pl.BlockSpec((3, 32), lambda i: (0, 0)),
            pl.BlockSpec((8, 16), lambda i: (0, 0)),
            pl.BlockSpec((8, 16), lambda i: (0, 0)),
            pl.BlockSpec((8, 64), lambda i: (0, 0)),
        ],
        out_specs=[
            pl.BlockSpec((PB, 128), lambda i: (i, 0)),
            pl.BlockSpec((PB, 48), lambda i: (i, 0)),
            pl.BlockSpec((8, 64), lambda i: (0, 0)),
        ],
        out_shape=[
            jax.ShapeDtypeStruct((N, 128), jnp.float32),
            jax.ShapeDtypeStruct((N, 48), jnp.float32),
            jax.ShapeDtypeStruct((8, 64), jnp.float32),
        ],
    )(coords, featT, vw, uw, f1w, b1row, wsT)


def _passA_body(g_ref, u_ref, d_ref, wall_ref, tst_ref, nf1_ref):
    i = pl.program_id(0)
    t3 = (g_ref[...][:, :, 0:48] + u_ref[...][:, None, :]
          + d_ref[...][:, :, None] * wall_ref[0:1, :][None, :, :])

    @pl.when(i == 0)
    def _():
        tst_ref[...] = jnp.zeros_like(tst_ref)

    s1 = jnp.sum(t3, axis=(0, 1))
    s2 = jnp.sum(t3 * t3, axis=(0, 1))
    pad = jnp.zeros((6, 48), jnp.float32)
    tst_ref[...] += jnp.concatenate([s1[None], s2[None], pad], axis=0)
    nf1_ref[...] = jnp.mean(t3[:, :, 32:48], axis=1)


def _passA(g3, u48, dist, wall):
    return pl.pallas_call(
        _passA_body,
        grid=(N // AB,),
        in_specs=[
            pl.BlockSpec((AB, KNB, 128), lambda i: (i, 0, 0)),
            pl.BlockSpec((AB, 48), lambda i: (i, 0)),
            pl.BlockSpec((AB, KNB), lambda i: (i, 0)),
            pl.BlockSpec((8, 48), lambda i: (0, 0)),
        ],
        out_specs=[
            pl.BlockSpec((8, 48), lambda i: (0, 0)),
            pl.BlockSpec((AB, 16), lambda i: (i, 0)),
        ],
        out_shape=[
            jax.ShapeDtypeStruct((8, 48), jnp.float32),
            jax.ShapeDtypeStruct((N, 16), jnp.float32),
        ],
    )(g3, u48, dist, wall)


def _passB1_body(g_ref, u_ref, d_ref, wall_ref, aff_ref, nf1_ref, wp1_ref,
                 y1_ref, mh2_ref, yst_ref):
    i = pl.program_id(0)
    t3 = (g_ref[...][:, :, 0:48] + u_ref[...][:, None, :]
          + d_ref[...][:, :, None] * wall_ref[0:1, :][None, :, :])
    h3 = t3 * aff_ref[0:1, :][None, :, :] + aff_ref[1:2, :][None, :, :]
    h3 = jnp.maximum(h3, 0.0)
    mh = jnp.mean(h3, axis=1)
    pooled1 = jnp.concatenate([mh[:, 0:16], nf1_ref[...]], axis=1)
    y1 = jnp.dot(pooled1, wp1_ref[...], preferred_element_type=jnp.float32)
    y1_ref[...] = y1
    mh2_ref[...] = mh[:, 16:32]

    @pl.when(i == 0)
    def _():
        yst_ref[...] = jnp.zeros_like(yst_ref)

    s1 = jnp.sum(y1, axis=0)
    s2 = jnp.sum(y1 * y1, axis=0)
    pad = jnp.zeros((6, 16), jnp.float32)
    yst_ref[...] += jnp.concatenate([s1[None], s2[None], pad], axis=0)


def _passB1(g3, u48, dist, wall, aff48, nf1, wp1T):
    return pl.pallas_call(
        _passB1_body,
        grid=(N // AB,),
        in_specs=[
            pl.BlockSpec((AB, KNB, 128), lambda i: (i, 0, 0)),
            pl.BlockSpec((AB, 48), lambda i: (i, 0)),
            pl.BlockSpec((AB, KNB), lambda i: (i, 0)),
            pl.BlockSpec((8, 48), lambda i: (0, 0)),
            pl.BlockSpec((8, 48), lambda i: (0, 0)),
            pl.BlockSpec((AB, 16), lambda i: (i, 0)),
            pl.BlockSpec((32, 16), lambda i: (0, 0)),
        ],
        out_specs=[
            pl.BlockSpec((AB, 16), lambda i: (i, 0)),
            pl.BlockSpec((AB, 16), lambda i: (i, 0)),
            pl.BlockSpec((8, 16), lambda i: (0, 0)),
        ],
        out_shape=[
            jax.ShapeDtypeStruct((N, 16), jnp.float32),
            jax.ShapeDtypeStruct((N, 16), jnp.float32),
            jax.ShapeDtypeStruct((8, 16), jnp.float32),
        ],
    )(g3, u48, dist, wall, aff48, nf1, wp1T)


def _passB3_body(mh2_ref, nf2_ref, wp2_ref, y2_ref, yst_ref):
    i = pl.program_id(0)
    pooled2 = jnp.concatenate([mh2_ref[...], nf2_ref[...]], axis=1)
    y2 = jnp.dot(pooled2, wp2_ref[...], preferred_element_type=jnp.float32)
    y2_ref[...] = y2

    @pl.when(i == 0)
    def _():
        yst_ref[...] = jnp.zeros_like(yst_ref)

    s1 = jnp.sum(y2, axis=0)
    s2 = jnp.sum(y2 * y2, axis=0)
    pad = jnp.zeros((6, 32), jnp.float32)
    yst_ref[...] += jnp.concatenate([s1[None], s2[None], pad], axis=0)


def _passB3(mh2, nf2, wp2T):
    return pl.pallas_call(
        _passB3_body,
        grid=(N // CB,),
        in_specs=[
            pl.BlockSpec((CB, 16), lambda i: (i, 0)),
            pl.BlockSpec((CB, 16), lambda i: (i, 0)),
            pl.BlockSpec((32, 32), lambda i: (0, 0)),
        ],
        out_specs=[
            pl.BlockSpec((CB, 32), lambda i: (i, 0)),
            pl.BlockSpec((8, 32), lambda i: (0, 0)),
        ],
        out_shape=[
            jax.ShapeDtypeStruct((N, 32), jnp.float32),
            jax.ShapeDtypeStruct((8, 32), jnp.float32),
        ],
    )(mh2, nf2, wp2T)


def _passB4_body(y2_ref, f8_ref, aff2_ref, ws_ref, w2_ref, saff_ref, b2_ref,
                 out_ref):
    y2 = y2_ref[...]
    x2 = jnp.maximum(y2 * aff2_ref[0:1, :] + aff2_ref[1:2, :], 0.0)
    ss = jnp.dot(ws_ref[...], f8_ref[...], preferred_element_type=jnp.float32)
    ssn = ss * saff_ref[:, 0:1] + saff_ref[:, 1:2]
    o = lax.dot_general(w2_ref[...], x2, (((1,), (1,)), ((), ())),
                        preferred_element_type=jnp.float32)
    o = o + b2_ref[...] + ssn
    out_ref[...] = jnp.where(o > 0, o, 0.01 * o)


def _passB4(y2, feat8, aff2, ws, w2, saff, b2col):
    return pl.pallas_call(
        _passB4_body,
        grid=(N // CB,),
        in_specs=[
            pl.BlockSpec((CB, 32), lambda i: (i, 0)),
            pl.BlockSpec((8, CB), lambda i: (0, i)),
            pl.BlockSpec((8, 32), lambda i: (0, 0)),
            pl.BlockSpec((64, 8), lambda i: (0, 0)),
            pl.BlockSpec((64, 32), lambda i: (0, 0)),
            pl.BlockSpec((64, 8), lambda i: (0, 0)),
            pl.BlockSpec((64, 1), lambda i: (0, 0)),
        ],
        out_specs=pl.BlockSpec((64, CB), lambda i: (0, i)),
        out_shape=jax.ShapeDtypeStruct((64, N), jnp.float32),
    )(y2, feat8, aff2, ws, w2, saff, b2col)


def _pad8(row):
    return jnp.concatenate([row[None, :], jnp.zeros((7, row.shape[0]), row.dtype)], 0)


def _stats_to_aff(s1, s2, count, gw, gb, group_size):
    C = s1.shape[0]
    g1 = s1.reshape(C // group_size, group_size).sum(1)
    g2 = s2.reshape(C // group_size, group_size).sum(1)
    m = g1 / (count * group_size)
    var = g2 / (count * group_size) - m * m
    inv = 1.0 / jnp.sqrt(var + EPS)
    m = jnp.repeat(m, group_size)
    inv = jnp.repeat(inv, group_size)
    scale = gw * inv
    shift = gb - m * scale
    return scale, shift


def kernel(coords, features, W1, b1, Wl1, bl1, gl1w, gl1b, Wp1, bp1, gp1w, gp1b, Wl2, bl2, gl2w, gl2b, Wp2, bp2, gp2w, gp2b, W2, b2, Ws, bs, gsw, gsb):
    c0 = coords[0]
    feat8 = features[0, :, :, 0]
    featT = feat8.T

    idxT, distT = _knn(c0)
    idx = idxT.T
    dist = distT.T
    idxf = idx.reshape(-1)

    A1 = Wl1[:, 0:3] + Wl1[:, 6:9]
    B1m = Wl1[:, 3:6] - Wl1[:, 6:9]
    w1 = Wl1[:, 9]
    A2 = Wl2[:, 0:3] + Wl2[:, 6:9]
    B2m = Wl2[:, 3:6] - Wl2[:, 6:9]
    w2 = Wl2[:, 9]
    vw = jnp.concatenate([B1m.T, B2m.T], axis=1)
    uw = jnp.concatenate([A1.T, A2.T], axis=1)
    wall = _pad8(jnp.concatenate([w1, w2, jnp.zeros(16, jnp.float32)]))

    T, U48, ssst = _p1(c0, featT, vw, uw, W1.T, _pad8(b1), Ws.T)

    g3 = _sc_gather_t(T, idxf).reshape(N, KNB, 128)

    tst, nf1 = _passA(g3, U48, dist, wall)

    gl12w = jnp.concatenate([gl1w, gl2w, jnp.zeros(16, jnp.float32)])
    gl12b = jnp.concatenate([gl1b, gl2b, jnp.zeros(16, jnp.float32)])
    sc48, sh48 = _stats_to_aff(tst[0], tst[1], float(NK), gl12w, gl12b, 1)
    aff48 = jnp.concatenate([sc48[None], sh48[None],
                             jnp.zeros((6, 48), jnp.float32)], 0)

    y1, mh2, y1st = _passB1(g3, U48, dist, wall, aff48, nf1, Wp1.T)

    sc1, shf1 = _stats_to_aff(y1st[0], y1st[1], float(N), gp1w, gp1b, 1)
    y1p = jnp.concatenate([y1, jnp.zeros((N, 112), jnp.float32)], axis=1)
    nf2 = _sc_gather_nf2(y1p, idxf, sc1, shf1)

    y2, y2st = _passB3(mh2, nf2, Wp2.T)

    sc2, shf2 = _stats_to_aff(y2st[0], y2st[1], float(N), gp2w, gp2b, 2)
    aff2 = jnp.concatenate([sc2[None], shf2[None],
                            jnp.zeros((6, 32), jnp.float32)], 0)
    scs, shs = _stats_to_aff(ssst[0], ssst[1], float(N), gsw, gsb, 4)
    saff = jnp.concatenate([scs[:, None], shs[:, None],
                            jnp.zeros((64, 6), jnp.float32)], 1)

    out = _passB4(y2, feat8, aff2, Ws, W2, saff, b2[:, None])
    return out[None, :, :, None]
